# Initial kernel scaffold; baseline (speedup 1.0000x reference)
import jax
import jax.numpy as jnp
from jax.experimental import pallas as pl


def kernel(nodes, edges, graph, node_norm_gamma, node_norm_beta, edge_norm_gamma, edge_norm_beta, nW1, nb1, nW2, nb2, eW1, eb1, eW2, eb2):
    raise NotImplementedError("write your pallas kernel here")


# speedup vs baseline: 2.9688x; 2.9688x over previous
import functools

import jax
import jax.numpy as jnp
from jax import lax
from jax.experimental import pallas as pl
from jax.experimental.pallas import tpu as pltpu
from jax.experimental.pallas import tpu_sc as plsc

_N = 10000
_E = 320000
_D = 128
_NC = 2
_NS = 16
_NW = _NC * _NS
_CW = 128
_NCHUNK = _E // _CW
_CPW = _NCHUNK // _NW
_REM = _NCHUNK - _CPW * _NW
_NP = 10240
_RPS = _NP // _NS
_EB = 2000
_NB = 1000


def _gelu(x):
    return 0.5 * x * (1.0 + lax.erf(x * 0.7071067811865476))


def _sc_scatter(edges, dst, z128):
    mesh = plsc.VectorSubcoreMesh(core_axis_name="c", subcore_axis_name="s")

    @functools.partial(
        pl.kernel,
        mesh=mesh,
        out_type=jax.ShapeDtypeStruct((_NC, _NP, _D), jnp.float32),
        scratch_types=[
            pltpu.VMEM((_CW, _D), jnp.float32),
            pltpu.VMEM((_CW,), jnp.int32),
            pltpu.VMEM_SHARED((_NP, _D), jnp.float32),
        ],
    )
    def k(edges_h, dst_h, z128_h, msgs_o, ebuf, didx, msgs_sh):
        cid = lax.axis_index("c")
        sid = lax.axis_index("s")
        wid = sid * _NC + cid
        r0 = sid * _RPS
        pltpu.sync_copy(z128_h.at[pl.ds(r0, _RPS)], msgs_sh.at[pl.ds(r0, _RPS)])
        plsc.subcore_barrier()

        def chunk(k_idx):
            e0 = k_idx * _CW
            pltpu.sync_copy(edges_h.at[pl.ds(e0, _CW)], ebuf)
            pltpu.sync_copy(dst_h.at[pl.ds(e0, _CW)], didx)
            pltpu.sync_copy(ebuf, msgs_sh.at[didx], add=True)

        def body(j, carry):
            chunk(wid * _CPW + j)
            return carry

        lax.fori_loop(0, _CPW, body, 0)

        @pl.when(wid < _REM)
        def _():
            chunk(_NW * _CPW + wid)

        plsc.subcore_barrier()
        pltpu.sync_copy(msgs_sh.at[pl.ds(r0, _RPS)],
                        msgs_o.at[cid, pl.ds(r0, _RPS)])

    return k(edges, dst, z128)


def _sc_counts(idx, z128, ones128):
    mesh = plsc.VectorSubcoreMesh(core_axis_name="c", subcore_axis_name="s")

    @functools.partial(
        pl.kernel,
        mesh=mesh,
        out_type=jax.ShapeDtypeStruct((_NC, _NP, _D), jnp.float32),
        scratch_types=[
            pltpu.VMEM((_CW,), jnp.int32),
            pltpu.VMEM((_CW, _D), jnp.float32),
            pltpu.VMEM_SHARED((_NP, _D), jnp.float32),
        ],
    )
    def k(idx_h, z128_h, ones_h, cnt_o, idxv, onesv, cnt_sh):
        cid = lax.axis_index("c")
        sid = lax.axis_index("s")
        wid = sid * _NC + cid
        r0 = sid * _RPS
        pltpu.sync_copy(z128_h.at[pl.ds(r0, _RPS)], cnt_sh.at[pl.ds(r0, _RPS)])
        pltpu.sync_copy(ones_h, onesv)
        plsc.subcore_barrier()

        def chunk(k_idx):
            e0 = k_idx * _CW
            pltpu.sync_copy(idx_h.at[pl.ds(e0, _CW)], idxv)
            pltpu.sync_copy(onesv, cnt_sh.at[idxv], add=True)

        def body(j, carry):
            chunk(wid * _CPW + j)
            return carry

        lax.fori_loop(0, _CPW, body, 0)

        @pl.when(wid < _REM)
        def _():
            chunk(_NW * _CPW + wid)

        plsc.subcore_barrier()
        pltpu.sync_copy(cnt_sh.at[pl.ds(r0, _RPS)],
                        cnt_o.at[cid, pl.ds(r0, _RPS)])

    return k(idx, z128, ones128)


def _sc_gather_add(p, q, src, dst):
    mesh = plsc.VectorSubcoreMesh(core_axis_name="c", subcore_axis_name="s")

    @functools.partial(
        pl.kernel,
        mesh=mesh,
        out_type=jax.ShapeDtypeStruct((_E, _D), jnp.float32),
        scratch_types=[
            pltpu.VMEM((_CW,), jnp.int32),
            pltpu.VMEM((_CW,), jnp.int32),
            pltpu.VMEM((_CW, _D), jnp.float32),
            pltpu.VMEM((_CW, _D), jnp.float32),
            pltpu.SemaphoreType.DMA,
        ],
    )
    def k(p_h, q_h, src_h, dst_h, r_o, sidx, didx, pbuf, qbuf, sem):
        cid = lax.axis_index("c")
        sid = lax.axis_index("s")
        wid = sid * _NC + cid

        def chunk(k_idx):
            e0 = k_idx * _CW
            pltpu.sync_copy(src_h.at[pl.ds(e0, _CW)], sidx)
            pltpu.sync_copy(dst_h.at[pl.ds(e0, _CW)], didx)
            pltpu.async_copy(p_h.at[sidx], pbuf, sem).wait()
            pltpu.async_copy(q_h.at[didx], qbuf, sem).wait()

            def row(r, c2):
                for t in range(_D // 16):
                    sl = pl.ds(t * 16, 16)
                    pbuf[r, sl] = pbuf[r, sl] + qbuf[r, sl]
                return c2

            lax.fori_loop(0, _CW, row, 0)
            pltpu.sync_copy(pbuf, r_o.at[pl.ds(e0, _CW)])

        def body(j, carry):
            chunk(wid * _CPW + j)
            return carry

        lax.fori_loop(0, _CPW, body, 0)

        @pl.when(wid < _REM)
        def _():
            chunk(_NW * _CPW + wid)

    return k(p, q, src, dst)


def _esq_body(e_ref, o_ref, acc):
    i = pl.program_id(0)

    @pl.when(i == 0)
    def _():
        acc[...] = jnp.zeros_like(acc)

    x = e_ref[...]
    acc[0:1, :] += jnp.sum(x * x, axis=0, keepdims=True)

    @pl.when(i == pl.num_programs(0) - 1)
    def _():
        o_ref[...] = acc[...]


def _nstats_body(n_ref, mp_ref, msgs_ref, st_ref, acc):
    i = pl.program_id(0)

    @pl.when(i == 0)
    def _():
        acc[...] = jnp.zeros_like(acc)

    x = n_ref[...]
    m = mp_ref[0] + mp_ref[1]
    msgs_ref[...] = m
    acc[0:1, :] += jnp.sum(x, axis=0, keepdims=True)
    acc[1:2, :] += jnp.sum(m, axis=0, keepdims=True)
    acc[2:3, :] += jnp.sum(x * x, axis=0, keepdims=True)
    acc[3:4, :] += jnp.sum(m * m, axis=0, keepdims=True)

    @pl.when(i == pl.num_programs(0) - 1)
    def _():
        st_ref[...] = acc[...]


def _node_body(n_ref, m_ref, w1a_ref, w1b_ref, b1_ref, w2_ref, b2_ref,
               cs_ref, cd_ref, nn_ref, ws_ref, acc):
    i = pl.program_id(0)

    @pl.when(i == 0)
    def _():
        acc[...] = jnp.zeros_like(acc)

    x = n_ref[...]
    m = m_ref[...]
    z = (jnp.dot(x, w1a_ref[...], preferred_element_type=jnp.float32)
         + jnp.dot(m, w1b_ref[...], preferred_element_type=jnp.float32)
         + b1_ref[...])
    h = _gelu(z)
    nn = jnp.dot(h, w2_ref[...], preferred_element_type=jnp.float32) \
        + b2_ref[...] + x
    nn_ref[...] = nn
    nn2 = nn * nn
    cs = cs_ref[...]
    cd = cd_ref[...]
    dn = (((0,), (0,)), ((), ()))
    acc[0:1, :] += lax.dot_general(cs, nn, dn,
                                   preferred_element_type=jnp.float32)
    acc[1:2, :] += lax.dot_general(cs, nn2, dn,
                                   preferred_element_type=jnp.float32)
    acc[2:3, :] += lax.dot_general(cd, nn, dn,
                                   preferred_element_type=jnp.float32)
    acc[3:4, :] += lax.dot_general(cd, nn2, dn,
                                   preferred_element_type=jnp.float32)

    @pl.when(i == pl.num_programs(0) - 1)
    def _():
        ws_ref[...] = acc[...]


def _pq_body(n_ref, a_ref, b_ref, p_ref, q_ref):
    x = n_ref[...]
    p_ref[...] = jnp.dot(x, a_ref[...], preferred_element_type=jnp.float32)
    q_ref[...] = jnp.dot(x, b_ref[...], preferred_element_type=jnp.float32)


def _edge_body(e_ref, r_ref, c_ref, b1_ref, w2_ref, b2_ref, o_ref):
    e = e_ref[...]
    z = jnp.dot(e, c_ref[...], preferred_element_type=jnp.float32) \
        + r_ref[...] + b1_ref[...]
    h = _gelu(z)
    o_ref[...] = jnp.dot(h, w2_ref[...], preferred_element_type=jnp.float32) \
        + b2_ref[...] + e


def _row_spec(shape):
    return pl.BlockSpec(shape, lambda i: (0,) * len(shape))


def kernel(nodes, edges, graph, node_norm_gamma, node_norm_beta,
           edge_norm_gamma, edge_norm_beta,
           nW1, nb1, nW2, nb2, eW1, eb1, eW2, eb2):
    f32 = jnp.float32
    src = graph[0]
    dst = graph[1]
    z128 = jnp.zeros((_NP, _D), f32)
    ones128 = jnp.ones((_CW, _D), f32)

    msgs_p = _sc_scatter(edges, dst, z128)
    csrc_p = _sc_counts(src, z128, ones128)
    cdst_p = _sc_counts(dst, z128, ones128)

    sumsq_e = pl.pallas_call(
        _esq_body,
        grid=(_E // _EB,),
        in_specs=[pl.BlockSpec((_EB, _D), lambda i: (i, 0))],
        out_specs=_row_spec((8, _D)),
        out_shape=jax.ShapeDtypeStruct((8, _D), f32),
        scratch_shapes=[pltpu.VMEM((8, _D), f32)],
    )(edges)[0]

    msgs, nst = pl.pallas_call(
        _nstats_body,
        grid=(_N // _NB,),
        in_specs=[
            pl.BlockSpec((_NB, _D), lambda i: (i, 0)),
            pl.BlockSpec((_NC, _NB, _D), lambda i: (0, i, 0)),
        ],
        out_specs=[
            pl.BlockSpec((_NB, _D), lambda i: (i, 0)),
            _row_spec((8, _D)),
        ],
        out_shape=[
            jax.ShapeDtypeStruct((_N, _D), f32),
            jax.ShapeDtypeStruct((8, _D), f32),
        ],
        scratch_shapes=[pltpu.VMEM((8, _D), f32)],
    )(nodes, msgs_p)

    mean_n = jnp.concatenate([nst[0], nst[1]]) / _N
    ex2_n = jnp.concatenate([nst[2], nst[3]]) / _N
    var_n = ex2_n - mean_n * mean_n
    scale_n = node_norm_gamma / jnp.sqrt(var_n + 1e-5)
    shift_n = node_norm_beta - mean_n * scale_n
    w1f = nW1 * scale_n[:, None]
    b1f = (nb1 + shift_n @ nW1).reshape(1, _D)

    csrc = (csrc_p[0, :_N, 0:1] + csrc_p[1, :_N, 0:1])
    cdst = (cdst_p[0, :_N, 0:1] + cdst_p[1, :_N, 0:1])

    nodes_new, ws = pl.pallas_call(
        _node_body,
        grid=(_N // _NB,),
        in_specs=[
            pl.BlockSpec((_NB, _D), lambda i: (i, 0)),
            pl.BlockSpec((_NB, _D), lambda i: (i, 0)),
            _row_spec((_D, _D)),
            _row_spec((_D, _D)),
            _row_spec((1, _D)),
            _row_spec((_D, _D)),
            _row_spec((1, _D)),
            pl.BlockSpec((_NB, 1), lambda i: (i, 0)),
            pl.BlockSpec((_NB, 1), lambda i: (i, 0)),
        ],
        out_specs=[
            pl.BlockSpec((_NB, _D), lambda i: (i, 0)),
            _row_spec((8, _D)),
        ],
        out_shape=[
            jax.ShapeDtypeStruct((_N, _D), f32),
            jax.ShapeDtypeStruct((8, _D), f32),
        ],
        scratch_shapes=[pltpu.VMEM((8, _D), f32)],
    )(nodes, msgs, w1f[:_D], w1f[_D:], b1f, nW2, nb2.reshape(1, _D),
      csrc, cdst)

    mean_e = jnp.concatenate([ws[0], ws[2], nst[1]]) / _E
    ex2_e = jnp.concatenate([ws[1], ws[3], sumsq_e]) / _E
    var_e = ex2_e - mean_e * mean_e
    scale_e = edge_norm_gamma / jnp.sqrt(var_e + 1e-5)
    shift_e = edge_norm_beta - mean_e * scale_e
    w1fe = eW1 * scale_e[:, None]
    b1fe = (eb1 + shift_e @ eW1).reshape(1, _D)

    p, q = pl.pallas_call(
        _pq_body,
        grid=(_N // _NB,),
        in_specs=[
            pl.BlockSpec((_NB, _D), lambda i: (i, 0)),
            _row_spec((_D, _D)),
            _row_spec((_D, _D)),
        ],
        out_specs=[
            pl.BlockSpec((_NB, _D), lambda i: (i, 0)),
            pl.BlockSpec((_NB, _D), lambda i: (i, 0)),
        ],
        out_shape=[
            jax.ShapeDtypeStruct((_N, _D), f32),
            jax.ShapeDtypeStruct((_N, _D), f32),
        ],
    )(nodes_new, w1fe[:_D], w1fe[_D:2 * _D])

    r = _sc_gather_add(p, q, src, dst)

    edges_new = pl.pallas_call(
        _edge_body,
        grid=(_E // _EB,),
        in_specs=[
            pl.BlockSpec((_EB, _D), lambda i: (i, 0)),
            pl.BlockSpec((_EB, _D), lambda i: (i, 0)),
            _row_spec((_D, _D)),
            _row_spec((1, _D)),
            _row_spec((_D, _D)),
            _row_spec((1, _D)),
        ],
        out_specs=pl.BlockSpec((_EB, _D), lambda i: (i, 0)),
        out_shape=jax.ShapeDtypeStruct((_E, _D), f32),
    )(edges, r, w1fe[2 * _D:], b1fe, eW2, eb2.reshape(1, _D))

    return (nodes_new, edges_new)


# speedup vs baseline: 3.3551x; 1.1301x over previous
import functools

import jax
import jax.numpy as jnp
from jax import lax
from jax.experimental import pallas as pl
from jax.experimental.pallas import tpu as pltpu
from jax.experimental.pallas import tpu_sc as plsc

_N = 10000
_E = 320000
_D = 128
_NC = 2
_NS = 16
_NW = _NC * _NS
_CW = 128
_NCHUNK = _E // _CW
_CPW = _NCHUNK // _NW
_REM = _NCHUNK - _CPW * _NW
_NP = 10240
_RPS = _NP // _NS
_EB = 2000
_NB = 1000


def _gelu(x):
    return 0.5 * x * (1.0 + lax.erf(x * 0.7071067811865476))


def _sc_scatter(edges, dst, z128):
    mesh = plsc.VectorSubcoreMesh(core_axis_name="c", subcore_axis_name="s")

    @functools.partial(
        pl.kernel,
        mesh=mesh,
        out_type=jax.ShapeDtypeStruct((_NC, _NP, _D), jnp.float32),
        scratch_types=[
            pltpu.VMEM((_CW, _D), jnp.float32),
            pltpu.VMEM((_CW,), jnp.int32),
            pltpu.VMEM_SHARED((_NP, _D), jnp.float32),
        ],
    )
    def k(edges_h, dst_h, z128_h, msgs_o, ebuf, didx, msgs_sh):
        cid = lax.axis_index("c")
        sid = lax.axis_index("s")
        wid = sid * _NC + cid
        r0 = sid * _RPS
        pltpu.sync_copy(z128_h.at[pl.ds(r0, _RPS)], msgs_sh.at[pl.ds(r0, _RPS)])
        plsc.subcore_barrier()

        def chunk(k_idx):
            e0 = k_idx * _CW
            pltpu.sync_copy(edges_h.at[pl.ds(e0, _CW)], ebuf)
            pltpu.sync_copy(dst_h.at[pl.ds(e0, _CW)], didx)
            pltpu.sync_copy(ebuf, msgs_sh.at[didx], add=True)

        def body(j, carry):
            chunk(wid * _CPW + j)
            return carry

        lax.fori_loop(0, _CPW, body, 0)

        @pl.when(wid < _REM)
        def _():
            chunk(_NW * _CPW + wid)

        plsc.subcore_barrier()
        pltpu.sync_copy(msgs_sh.at[pl.ds(r0, _RPS)],
                        msgs_o.at[cid, pl.ds(r0, _RPS)])

    return k(edges, dst, z128)


def _sc_counts(idx, z128, ones128):
    mesh = plsc.VectorSubcoreMesh(core_axis_name="c", subcore_axis_name="s")

    @functools.partial(
        pl.kernel,
        mesh=mesh,
        out_type=jax.ShapeDtypeStruct((_NC, _NP, _D), jnp.float32),
        scratch_types=[
            pltpu.VMEM((_CW,), jnp.int32),
            pltpu.VMEM((_CW, _D), jnp.float32),
            pltpu.VMEM_SHARED((_NP, _D), jnp.float32),
        ],
    )
    def k(idx_h, z128_h, ones_h, cnt_o, idxv, onesv, cnt_sh):
        cid = lax.axis_index("c")
        sid = lax.axis_index("s")
        wid = sid * _NC + cid
        r0 = sid * _RPS
        pltpu.sync_copy(z128_h.at[pl.ds(r0, _RPS)], cnt_sh.at[pl.ds(r0, _RPS)])
        pltpu.sync_copy(ones_h, onesv)
        plsc.subcore_barrier()

        def chunk(k_idx):
            e0 = k_idx * _CW
            pltpu.sync_copy(idx_h.at[pl.ds(e0, _CW)], idxv)
            pltpu.sync_copy(onesv, cnt_sh.at[idxv], add=True)

        def body(j, carry):
            chunk(wid * _CPW + j)
            return carry

        lax.fori_loop(0, _CPW, body, 0)

        @pl.when(wid < _REM)
        def _():
            chunk(_NW * _CPW + wid)

        plsc.subcore_barrier()
        pltpu.sync_copy(cnt_sh.at[pl.ds(r0, _RPS)],
                        cnt_o.at[cid, pl.ds(r0, _RPS)])

    return k(idx, z128, ones128)


def _sc_gather_add(p, q, src, dst):
    mesh = plsc.VectorSubcoreMesh(core_axis_name="c", subcore_axis_name="s")

    @functools.partial(
        pl.kernel,
        mesh=mesh,
        out_type=jax.ShapeDtypeStruct((_E, _D), jnp.float32),
        scratch_types=[
            pltpu.VMEM((2, _CW), jnp.int32),
            pltpu.VMEM((2, _CW), jnp.int32),
            pltpu.VMEM((2, _CW, _D), jnp.float32),
            pltpu.VMEM((2, _CW, _D), jnp.float32),
            pltpu.SemaphoreType.DMA,
            pltpu.SemaphoreType.DMA,
            pltpu.SemaphoreType.DMA,
        ],
    )
    def k(p_h, q_h, src_h, dst_h, r_o, sidx, didx, pbuf, qbuf,
          semi, semg, sems):
        cid = lax.axis_index("c")
        sid = lax.axis_index("s")
        wid = sid * _NC + cid

        def add_rows(b):
            def row(r, c2):
                for t in range(_D // 16):
                    sl = pl.ds(t * 16, 16)
                    plsc.addupdate(pbuf.at[b, r, sl], qbuf[b, r, sl])
                return c2

            lax.fori_loop(0, _CW, row, 0)

        def group(t, carry):
            j0 = (wid * _CPW + 2 * t) * _CW
            hs = []
            for b in range(2):
                e0 = j0 + b * _CW
                hs.append(pltpu.async_copy(src_h.at[pl.ds(e0, _CW)],
                                           sidx.at[b], semi))
                hs.append(pltpu.async_copy(dst_h.at[pl.ds(e0, _CW)],
                                           didx.at[b], semi))
            for h in hs:
                h.wait()
            gs = []
            for b in range(2):
                gs.append(pltpu.async_copy(p_h.at[sidx.at[b]],
                                           pbuf.at[b], semg))
                gs.append(pltpu.async_copy(q_h.at[didx.at[b]],
                                           qbuf.at[b], semg))
            for h in gs:
                h.wait()
            for b in range(2):
                add_rows(b)
            ss = []
            for b in range(2):
                e0 = j0 + b * _CW
                ss.append(pltpu.async_copy(pbuf.at[b],
                                           r_o.at[pl.ds(e0, _CW)], sems))
            for h in ss:
                h.wait()
            return carry

        lax.fori_loop(0, _CPW // 2, group, 0)

        @pl.when(wid < _REM)
        def _():
            e0 = (_NW * _CPW + wid) * _CW
            pltpu.sync_copy(src_h.at[pl.ds(e0, _CW)], sidx.at[0])
            pltpu.sync_copy(dst_h.at[pl.ds(e0, _CW)], didx.at[0])
            pltpu.async_copy(p_h.at[sidx.at[0]], pbuf.at[0], semg).wait()
            pltpu.async_copy(q_h.at[didx.at[0]], qbuf.at[0], semg).wait()
            add_rows(0)
            pltpu.sync_copy(pbuf.at[0], r_o.at[pl.ds(e0, _CW)])

    return k(p, q, src, dst)


def _esq_body(e_ref, o_ref, acc):
    i = pl.program_id(0)

    @pl.when(i == 0)
    def _():
        acc[...] = jnp.zeros_like(acc)

    x = e_ref[...]
    acc[0:1, :] += jnp.sum(x * x, axis=0, keepdims=True)

    @pl.when(i == pl.num_programs(0) - 1)
    def _():
        o_ref[...] = acc[...]


def _nstats_body(n_ref, mp_ref, msgs_ref, st_ref, acc):
    i = pl.program_id(0)

    @pl.when(i == 0)
    def _():
        acc[...] = jnp.zeros_like(acc)

    x = n_ref[...]
    m = mp_ref[0] + mp_ref[1]
    msgs_ref[...] = m
    acc[0:1, :] += jnp.sum(x, axis=0, keepdims=True)
    acc[1:2, :] += jnp.sum(m, axis=0, keepdims=True)
    acc[2:3, :] += jnp.sum(x * x, axis=0, keepdims=True)
    acc[3:4, :] += jnp.sum(m * m, axis=0, keepdims=True)

    @pl.when(i == pl.num_programs(0) - 1)
    def _():
        st_ref[...] = acc[...]


def _node_body(n_ref, m_ref, w1a_ref, w1b_ref, b1_ref, w2_ref, b2_ref,
               cs_ref, cd_ref, nn_ref, ws_ref, acc):
    i = pl.program_id(0)

    @pl.when(i == 0)
    def _():
        acc[...] = jnp.zeros_like(acc)

    x = n_ref[...]
    m = m_ref[...]
    z = (jnp.dot(x, w1a_ref[...], preferred_element_type=jnp.float32)
         + jnp.dot(m, w1b_ref[...], preferred_element_type=jnp.float32)
         + b1_ref[...])
    h = _gelu(z)
    nn = jnp.dot(h, w2_ref[...], preferred_element_type=jnp.float32) \
        + b2_ref[...] + x
    nn_ref[...] = nn
    nn2 = nn * nn
    cs = cs_ref[...]
    cd = cd_ref[...]
    dn = (((0,), (0,)), ((), ()))
    acc[0:1, :] += lax.dot_general(cs, nn, dn,
                                   preferred_element_type=jnp.float32)
    acc[1:2, :] += lax.dot_general(cs, nn2, dn,
                                   preferred_element_type=jnp.float32)
    acc[2:3, :] += lax.dot_general(cd, nn, dn,
                                   preferred_element_type=jnp.float32)
    acc[3:4, :] += lax.dot_general(cd, nn2, dn,
                                   preferred_element_type=jnp.float32)

    @pl.when(i == pl.num_programs(0) - 1)
    def _():
        ws_ref[...] = acc[...]


def _pq_body(n_ref, a_ref, b_ref, p_ref, q_ref):
    x = n_ref[...]
    p_ref[...] = jnp.dot(x, a_ref[...], preferred_element_type=jnp.float32)
    q_ref[...] = jnp.dot(x, b_ref[...], preferred_element_type=jnp.float32)


def _edge_body(e_ref, r_ref, c_ref, b1_ref, w2_ref, b2_ref, o_ref):
    e = e_ref[...]
    z = jnp.dot(e, c_ref[...], preferred_element_type=jnp.float32) \
        + r_ref[...] + b1_ref[...]
    h = _gelu(z)
    o_ref[...] = jnp.dot(h, w2_ref[...], preferred_element_type=jnp.float32) \
        + b2_ref[...] + e


def _row_spec(shape):
    return pl.BlockSpec(shape, lambda i: (0,) * len(shape))


def kernel(nodes, edges, graph, node_norm_gamma, node_norm_beta,
           edge_norm_gamma, edge_norm_beta,
           nW1, nb1, nW2, nb2, eW1, eb1, eW2, eb2):
    f32 = jnp.float32
    src = graph[0]
    dst = graph[1]
    z128 = jnp.zeros((_NP, _D), f32)
    ones128 = jnp.ones((_CW, _D), f32)

    msgs_p = _sc_scatter(edges, dst, z128)
    csrc_p = _sc_counts(src, z128, ones128)
    cdst_p = _sc_counts(dst, z128, ones128)

    sumsq_e = pl.pallas_call(
        _esq_body,
        grid=(_E // _EB,),
        in_specs=[pl.BlockSpec((_EB, _D), lambda i: (i, 0))],
        out_specs=_row_spec((8, _D)),
        out_shape=jax.ShapeDtypeStruct((8, _D), f32),
        scratch_shapes=[pltpu.VMEM((8, _D), f32)],
    )(edges)[0]

    msgs, nst = pl.pallas_call(
        _nstats_body,
        grid=(_N // _NB,),
        in_specs=[
            pl.BlockSpec((_NB, _D), lambda i: (i, 0)),
            pl.BlockSpec((_NC, _NB, _D), lambda i: (0, i, 0)),
        ],
        out_specs=[
            pl.BlockSpec((_NB, _D), lambda i: (i, 0)),
            _row_spec((8, _D)),
        ],
        out_shape=[
            jax.ShapeDtypeStruct((_N, _D), f32),
            jax.ShapeDtypeStruct((8, _D), f32),
        ],
        scratch_shapes=[pltpu.VMEM((8, _D), f32)],
    )(nodes, msgs_p)

    mean_n = jnp.concatenate([nst[0], nst[1]]) / _N
    ex2_n = jnp.concatenate([nst[2], nst[3]]) / _N
    var_n = ex2_n - mean_n * mean_n
    scale_n = node_norm_gamma / jnp.sqrt(var_n + 1e-5)
    shift_n = node_norm_beta - mean_n * scale_n
    w1f = nW1 * scale_n[:, None]
    b1f = (nb1 + shift_n @ nW1).reshape(1, _D)

    csrc = (csrc_p[0, :_N, 0:1] + csrc_p[1, :_N, 0:1])
    cdst = (cdst_p[0, :_N, 0:1] + cdst_p[1, :_N, 0:1])

    nodes_new, ws = pl.pallas_call(
        _node_body,
        grid=(_N // _NB,),
        in_specs=[
            pl.BlockSpec((_NB, _D), lambda i: (i, 0)),
            pl.BlockSpec((_NB, _D), lambda i: (i, 0)),
            _row_spec((_D, _D)),
            _row_spec((_D, _D)),
            _row_spec((1, _D)),
            _row_spec((_D, _D)),
            _row_spec((1, _D)),
            pl.BlockSpec((_NB, 1), lambda i: (i, 0)),
            pl.BlockSpec((_NB, 1), lambda i: (i, 0)),
        ],
        out_specs=[
            pl.BlockSpec((_NB, _D), lambda i: (i, 0)),
            _row_spec((8, _D)),
        ],
        out_shape=[
            jax.ShapeDtypeStruct((_N, _D), f32),
            jax.ShapeDtypeStruct((8, _D), f32),
        ],
        scratch_shapes=[pltpu.VMEM((8, _D), f32)],
    )(nodes, msgs, w1f[:_D], w1f[_D:], b1f, nW2, nb2.reshape(1, _D),
      csrc, cdst)

    mean_e = jnp.concatenate([ws[0], ws[2], nst[1]]) / _E
    ex2_e = jnp.concatenate([ws[1], ws[3], sumsq_e]) / _E
    var_e = ex2_e - mean_e * mean_e
    scale_e = edge_norm_gamma / jnp.sqrt(var_e + 1e-5)
    shift_e = edge_norm_beta - mean_e * scale_e
    w1fe = eW1 * scale_e[:, None]
    b1fe = (eb1 + shift_e @ eW1).reshape(1, _D)

    p, q = pl.pallas_call(
        _pq_body,
        grid=(_N // _NB,),
        in_specs=[
            pl.BlockSpec((_NB, _D), lambda i: (i, 0)),
            _row_spec((_D, _D)),
            _row_spec((_D, _D)),
        ],
        out_specs=[
            pl.BlockSpec((_NB, _D), lambda i: (i, 0)),
            pl.BlockSpec((_NB, _D), lambda i: (i, 0)),
        ],
        out_shape=[
            jax.ShapeDtypeStruct((_N, _D), f32),
            jax.ShapeDtypeStruct((_N, _D), f32),
        ],
    )(nodes_new, w1fe[:_D], w1fe[_D:2 * _D])

    r = _sc_gather_add(p, q, src, dst)

    edges_new = pl.pallas_call(
        _edge_body,
        grid=(_E // _EB,),
        in_specs=[
            pl.BlockSpec((_EB, _D), lambda i: (i, 0)),
            pl.BlockSpec((_EB, _D), lambda i: (i, 0)),
            _row_spec((_D, _D)),
            _row_spec((1, _D)),
            _row_spec((_D, _D)),
            _row_spec((1, _D)),
        ],
        out_specs=pl.BlockSpec((_EB, _D), lambda i: (i, 0)),
        out_shape=jax.ShapeDtypeStruct((_E, _D), f32),
    )(edges, r, w1fe[2 * _D:], b1fe, eW2, eb2.reshape(1, _D))

    return (nodes_new, edges_new)


# speedup vs baseline: 3.7768x; 1.1257x over previous
import functools

import jax
import jax.numpy as jnp
from jax import lax
from jax.experimental import pallas as pl
from jax.experimental.pallas import tpu as pltpu
from jax.experimental.pallas import tpu_sc as plsc

_N = 10000
_E = 320000
_D = 128
_NC = 2
_NS = 16
_NW = _NC * _NS
_CW = 128
_NCHUNK = _E // _CW
_CPW = _NCHUNK // _NW
_REM = _NCHUNK - _CPW * _NW
_NP = 10240
_RPS = _NP // _NS
_EB = 2000
_NB = 1000


def _gelu(x):
    return 0.5 * x * (1.0 + lax.erf(x * 0.7071067811865476))


def _sc_scatter(edges, dst, z128):
    mesh = plsc.VectorSubcoreMesh(core_axis_name="c", subcore_axis_name="s")

    @functools.partial(
        pl.kernel,
        mesh=mesh,
        out_type=jax.ShapeDtypeStruct((_NC, _NP, _D), jnp.float32),
        scratch_types=[
            pltpu.VMEM((_CW, _D), jnp.float32),
            pltpu.VMEM((_CW,), jnp.int32),
            pltpu.VMEM_SHARED((_NP, _D), jnp.float32),
        ],
    )
    def k(edges_h, dst_h, z128_h, msgs_o, ebuf, didx, msgs_sh):
        cid = lax.axis_index("c")
        sid = lax.axis_index("s")
        wid = sid * _NC + cid
        r0 = sid * _RPS
        pltpu.sync_copy(z128_h.at[pl.ds(r0, _RPS)], msgs_sh.at[pl.ds(r0, _RPS)])
        plsc.subcore_barrier()

        def chunk(k_idx):
            e0 = k_idx * _CW
            pltpu.sync_copy(edges_h.at[pl.ds(e0, _CW)], ebuf)
            pltpu.sync_copy(dst_h.at[pl.ds(e0, _CW)], didx)
            pltpu.sync_copy(ebuf, msgs_sh.at[didx], add=True)

        def body(j, carry):
            chunk(wid * _CPW + j)
            return carry

        lax.fori_loop(0, _CPW, body, 0)

        @pl.when(wid < _REM)
        def _():
            chunk(_NW * _CPW + wid)

        plsc.subcore_barrier()
        pltpu.sync_copy(msgs_sh.at[pl.ds(r0, _RPS)],
                        msgs_o.at[cid, pl.ds(r0, _RPS)])

    return k(edges, dst, z128)


def _sc_counts(src, dst, z1, ones1):
    mesh = plsc.VectorSubcoreMesh(core_axis_name="c", subcore_axis_name="s")

    @functools.partial(
        pl.kernel,
        mesh=mesh,
        out_type=[
            jax.ShapeDtypeStruct((_NC * _NP,), jnp.float32),
            jax.ShapeDtypeStruct((_NC * _NP,), jnp.float32),
        ],
        scratch_types=[
            pltpu.VMEM((_CW,), jnp.int32),
            pltpu.VMEM((_CW,), jnp.int32),
            pltpu.VMEM((_CW,), jnp.float32),
            pltpu.VMEM_SHARED((_NP,), jnp.float32),
            pltpu.VMEM_SHARED((_NP,), jnp.float32),
        ],
    )
    def k(src_h, dst_h, z1_h, ones_h, csrc_o, cdst_o,
          sidx, didx, onesv, csrc_sh, cdst_sh):
        cid = lax.axis_index("c")
        sid = lax.axis_index("s")
        wid = sid * _NC + cid
        r0 = sid * _RPS
        pltpu.sync_copy(z1_h.at[pl.ds(r0, _RPS)], csrc_sh.at[pl.ds(r0, _RPS)])
        pltpu.sync_copy(z1_h.at[pl.ds(r0, _RPS)], cdst_sh.at[pl.ds(r0, _RPS)])
        pltpu.sync_copy(ones_h, onesv)
        plsc.subcore_barrier()

        def chunk(k_idx):
            e0 = k_idx * _CW
            pltpu.sync_copy(src_h.at[pl.ds(e0, _CW)], sidx)
            pltpu.sync_copy(dst_h.at[pl.ds(e0, _CW)], didx)
            pltpu.sync_copy(onesv, csrc_sh.at[sidx], add=True)
            pltpu.sync_copy(onesv, cdst_sh.at[didx], add=True)

        def body(j, carry):
            chunk(wid * _CPW + j)
            return carry

        lax.fori_loop(0, _CPW, body, 0)

        @pl.when(wid < _REM)
        def _():
            chunk(_NW * _CPW + wid)

        plsc.subcore_barrier()
        pltpu.sync_copy(csrc_sh.at[pl.ds(r0, _RPS)],
                        csrc_o.at[pl.ds(cid * _NP + r0, _RPS)])
        pltpu.sync_copy(cdst_sh.at[pl.ds(r0, _RPS)],
                        cdst_o.at[pl.ds(cid * _NP + r0, _RPS)])

    return k(src, dst, z1, ones1)


def _sc_gather_add(p, q, src, dst):
    mesh = plsc.VectorSubcoreMesh(core_axis_name="c", subcore_axis_name="s")

    @functools.partial(
        pl.kernel,
        mesh=mesh,
        out_type=jax.ShapeDtypeStruct((_E, _D), jnp.float32),
        scratch_types=[
            pltpu.VMEM((2, _CW), jnp.int32),
            pltpu.VMEM((2, _CW), jnp.int32),
            pltpu.VMEM((2, _CW, _D), jnp.float32),
            pltpu.VMEM((2, _CW, _D), jnp.float32),
            pltpu.SemaphoreType.DMA,
            pltpu.SemaphoreType.DMA,
            pltpu.SemaphoreType.DMA,
        ],
    )
    def k(p_h, q_h, src_h, dst_h, r_o, sidx, didx, pbuf, qbuf,
          semi, semg, sems):
        cid = lax.axis_index("c")
        sid = lax.axis_index("s")
        wid = sid * _NC + cid

        def add_rows(b):
            def row(r, c2):
                for t in range(_D // 16):
                    sl = pl.ds(t * 16, 16)
                    plsc.addupdate(pbuf.at[b, r, sl], qbuf[b, r, sl])
                return c2

            lax.fori_loop(0, _CW, row, 0)

        def group(t, carry):
            j0 = (wid * _CPW + 2 * t) * _CW
            hs = []
            for b in range(2):
                e0 = j0 + b * _CW
                hs.append(pltpu.async_copy(src_h.at[pl.ds(e0, _CW)],
                                           sidx.at[b], semi))
                hs.append(pltpu.async_copy(dst_h.at[pl.ds(e0, _CW)],
                                           didx.at[b], semi))
            for h in hs:
                h.wait()
            gs = []
            for b in range(2):
                gs.append(pltpu.async_copy(p_h.at[sidx.at[b]],
                                           pbuf.at[b], semg))
                gs.append(pltpu.async_copy(q_h.at[didx.at[b]],
                                           qbuf.at[b], semg))
            for h in gs:
                h.wait()
            for b in range(2):
                add_rows(b)
            ss = []
            for b in range(2):
                e0 = j0 + b * _CW
                ss.append(pltpu.async_copy(pbuf.at[b],
                                           r_o.at[pl.ds(e0, _CW)], sems))
            for h in ss:
                h.wait()
            return carry

        lax.fori_loop(0, _CPW // 2, group, 0)

        @pl.when(wid < _REM)
        def _():
            e0 = (_NW * _CPW + wid) * _CW
            pltpu.sync_copy(src_h.at[pl.ds(e0, _CW)], sidx.at[0])
            pltpu.sync_copy(dst_h.at[pl.ds(e0, _CW)], didx.at[0])
            pltpu.async_copy(p_h.at[sidx.at[0]], pbuf.at[0], semg).wait()
            pltpu.async_copy(q_h.at[didx.at[0]], qbuf.at[0], semg).wait()
            add_rows(0)
            pltpu.sync_copy(pbuf.at[0], r_o.at[pl.ds(e0, _CW)])

    return k(p, q, src, dst)


def _esq_body(e_ref, o_ref, acc):
    i = pl.program_id(0)

    @pl.when(i == 0)
    def _():
        acc[...] = jnp.zeros_like(acc)

    x = e_ref[...]
    acc[0:1, :] += jnp.sum(x * x, axis=0, keepdims=True)

    @pl.when(i == pl.num_programs(0) - 1)
    def _():
        o_ref[...] = acc[...]


def _nstats_body(n_ref, mp_ref, msgs_ref, st_ref, acc):
    i = pl.program_id(0)

    @pl.when(i == 0)
    def _():
        acc[...] = jnp.zeros_like(acc)

    x = n_ref[...]
    m = mp_ref[0] + mp_ref[1]
    msgs_ref[...] = m
    acc[0:1, :] += jnp.sum(x, axis=0, keepdims=True)
    acc[1:2, :] += jnp.sum(m, axis=0, keepdims=True)
    acc[2:3, :] += jnp.sum(x * x, axis=0, keepdims=True)
    acc[3:4, :] += jnp.sum(m * m, axis=0, keepdims=True)

    @pl.when(i == pl.num_programs(0) - 1)
    def _():
        st_ref[...] = acc[...]


def _node_body(n_ref, m_ref, w1a_ref, w1b_ref, b1_ref, w2_ref, b2_ref,
               cs_ref, cd_ref, nn_ref, ws_ref, acc):
    i = pl.program_id(0)

    @pl.when(i == 0)
    def _():
        acc[...] = jnp.zeros_like(acc)

    x = n_ref[...]
    m = m_ref[...]
    z = (jnp.dot(x, w1a_ref[...], preferred_element_type=jnp.float32)
         + jnp.dot(m, w1b_ref[...], preferred_element_type=jnp.float32)
         + b1_ref[...])
    h = _gelu(z)
    nn = jnp.dot(h, w2_ref[...], preferred_element_type=jnp.float32) \
        + b2_ref[...] + x
    nn_ref[...] = nn
    nn2 = nn * nn
    cs = cs_ref[...]
    cd = cd_ref[...]
    dn = (((0,), (0,)), ((), ()))
    acc[0:1, :] += lax.dot_general(cs, nn, dn,
                                   preferred_element_type=jnp.float32)
    acc[1:2, :] += lax.dot_general(cs, nn2, dn,
                                   preferred_element_type=jnp.float32)
    acc[2:3, :] += lax.dot_general(cd, nn, dn,
                                   preferred_element_type=jnp.float32)
    acc[3:4, :] += lax.dot_general(cd, nn2, dn,
                                   preferred_element_type=jnp.float32)

    @pl.when(i == pl.num_programs(0) - 1)
    def _():
        ws_ref[...] = acc[...]


def _pq_body(n_ref, a_ref, b_ref, p_ref, q_ref):
    x = n_ref[...]
    p_ref[...] = jnp.dot(x, a_ref[...], preferred_element_type=jnp.float32)
    q_ref[...] = jnp.dot(x, b_ref[...], preferred_element_type=jnp.float32)


def _edge_body(e_ref, r_ref, c_ref, b1_ref, w2_ref, b2_ref, o_ref):
    e = e_ref[...]
    z = jnp.dot(e, c_ref[...], preferred_element_type=jnp.float32) \
        + r_ref[...] + b1_ref[...]
    h = _gelu(z)
    o_ref[...] = jnp.dot(h, w2_ref[...], preferred_element_type=jnp.float32) \
        + b2_ref[...] + e


def _row_spec(shape):
    return pl.BlockSpec(shape, lambda i: (0,) * len(shape))


def kernel(nodes, edges, graph, node_norm_gamma, node_norm_beta,
           edge_norm_gamma, edge_norm_beta,
           nW1, nb1, nW2, nb2, eW1, eb1, eW2, eb2):
    f32 = jnp.float32
    src = graph[0]
    dst = graph[1]
    z128 = jnp.zeros((_NP, _D), f32)
    z1 = jnp.zeros((_NP,), f32)
    ones1 = jnp.ones((_CW,), f32)

    msgs_p = _sc_scatter(edges, dst, z128)
    csrc_1, cdst_1 = _sc_counts(src, dst, z1, ones1)

    sumsq_e = pl.pallas_call(
        _esq_body,
        grid=(_E // _EB,),
        in_specs=[pl.BlockSpec((_EB, _D), lambda i: (i, 0))],
        out_specs=_row_spec((8, _D)),
        out_shape=jax.ShapeDtypeStruct((8, _D), f32),
        scratch_shapes=[pltpu.VMEM((8, _D), f32)],
    )(edges)[0]

    msgs, nst = pl.pallas_call(
        _nstats_body,
        grid=(_N // _NB,),
        in_specs=[
            pl.BlockSpec((_NB, _D), lambda i: (i, 0)),
            pl.BlockSpec((_NC, _NB, _D), lambda i: (0, i, 0)),
        ],
        out_specs=[
            pl.BlockSpec((_NB, _D), lambda i: (i, 0)),
            _row_spec((8, _D)),
        ],
        out_shape=[
            jax.ShapeDtypeStruct((_N, _D), f32),
            jax.ShapeDtypeStruct((8, _D), f32),
        ],
        scratch_shapes=[pltpu.VMEM((8, _D), f32)],
    )(nodes, msgs_p)

    mean_n = jnp.concatenate([nst[0], nst[1]]) / _N
    ex2_n = jnp.concatenate([nst[2], nst[3]]) / _N
    var_n = ex2_n - mean_n * mean_n
    scale_n = node_norm_gamma / jnp.sqrt(var_n + 1e-5)
    shift_n = node_norm_beta - mean_n * scale_n
    w1f = nW1 * scale_n[:, None]
    b1f = (nb1 + shift_n @ nW1).reshape(1, _D)

    csrc = (csrc_1[:_N] + csrc_1[_NP:_NP + _N]).reshape(_N, 1)
    cdst = (cdst_1[:_N] + cdst_1[_NP:_NP + _N]).reshape(_N, 1)

    nodes_new, ws = pl.pallas_call(
        _node_body,
        grid=(_N // _NB,),
        in_specs=[
            pl.BlockSpec((_NB, _D), lambda i: (i, 0)),
            pl.BlockSpec((_NB, _D), lambda i: (i, 0)),
            _row_spec((_D, _D)),
            _row_spec((_D, _D)),
            _row_spec((1, _D)),
            _row_spec((_D, _D)),
            _row_spec((1, _D)),
            pl.BlockSpec((_NB, 1), lambda i: (i, 0)),
            pl.BlockSpec((_NB, 1), lambda i: (i, 0)),
        ],
        out_specs=[
            pl.BlockSpec((_NB, _D), lambda i: (i, 0)),
            _row_spec((8, _D)),
        ],
        out_shape=[
            jax.ShapeDtypeStruct((_N, _D), f32),
            jax.ShapeDtypeStruct((8, _D), f32),
        ],
        scratch_shapes=[pltpu.VMEM((8, _D), f32)],
    )(nodes, msgs, w1f[:_D], w1f[_D:], b1f, nW2, nb2.reshape(1, _D),
      csrc, cdst)

    mean_e = jnp.concatenate([ws[0], ws[2], nst[1]]) / _E
    ex2_e = jnp.concatenate([ws[1], ws[3], sumsq_e]) / _E
    var_e = ex2_e - mean_e * mean_e
    scale_e = edge_norm_gamma / jnp.sqrt(var_e + 1e-5)
    shift_e = edge_norm_beta - mean_e * scale_e
    w1fe = eW1 * scale_e[:, None]
    b1fe = (eb1 + shift_e @ eW1).reshape(1, _D)

    p, q = pl.pallas_call(
        _pq_body,
        grid=(_N // _NB,),
        in_specs=[
            pl.BlockSpec((_NB, _D), lambda i: (i, 0)),
            _row_spec((_D, _D)),
            _row_spec((_D, _D)),
        ],
        out_specs=[
            pl.BlockSpec((_NB, _D), lambda i: (i, 0)),
            pl.BlockSpec((_NB, _D), lambda i: (i, 0)),
        ],
        out_shape=[
            jax.ShapeDtypeStruct((_N, _D), f32),
            jax.ShapeDtypeStruct((_N, _D), f32),
        ],
    )(nodes_new, w1fe[:_D], w1fe[_D:2 * _D])

    r = _sc_gather_add(p, q, src, dst)

    edges_new = pl.pallas_call(
        _edge_body,
        grid=(_E // _EB,),
        in_specs=[
            pl.BlockSpec((_EB, _D), lambda i: (i, 0)),
            pl.BlockSpec((_EB, _D), lambda i: (i, 0)),
            _row_spec((_D, _D)),
            _row_spec((1, _D)),
            _row_spec((_D, _D)),
            _row_spec((1, _D)),
        ],
        out_specs=pl.BlockSpec((_EB, _D), lambda i: (i, 0)),
        out_shape=jax.ShapeDtypeStruct((_E, _D), f32),
    )(edges, r, w1fe[2 * _D:], b1fe, eW2, eb2.reshape(1, _D))

    return (nodes_new, edges_new)


# speedup vs baseline: 4.0426x; 1.0704x over previous
import functools

import jax
import jax.numpy as jnp
from jax import lax
from jax.experimental import pallas as pl
from jax.experimental.pallas import tpu as pltpu
from jax.experimental.pallas import tpu_sc as plsc

_N = 10000
_E = 320000
_D = 128
_NC = 2
_NS = 16
_NW = _NC * _NS
_CW = 128
_NCHUNK = _E // _CW
_CPW = _NCHUNK // _NW
_REM = _NCHUNK - _CPW * _NW
_NP = 10240
_RPS = _NP // _NS
_EB = 2000
_NB = 1000


def _gelu(x):
    return 0.5 * x * (1.0 + lax.erf(x * 0.7071067811865476))


def _sc_scatter(edges, dst, z128):
    mesh = plsc.VectorSubcoreMesh(core_axis_name="c", subcore_axis_name="s")

    @functools.partial(
        pl.kernel,
        mesh=mesh,
        out_type=jax.ShapeDtypeStruct((_NC, _NP, _D), jnp.float32),
        scratch_types=[
            pltpu.VMEM((2, _CW, _D), jnp.float32),
            pltpu.VMEM((2, _CW), jnp.int32),
            pltpu.VMEM_SHARED((_NP, _D), jnp.float32),
            pltpu.SemaphoreType.DMA,
            pltpu.SemaphoreType.DMA,
        ],
    )
    def k(edges_h, dst_h, z128_h, msgs_o, ebuf, didx, msgs_sh, semi, semsc):
        cid = lax.axis_index("c")
        sid = lax.axis_index("s")
        wid = sid * _NC + cid
        r0 = sid * _RPS
        pltpu.sync_copy(z128_h.at[pl.ds(r0, _RPS)], msgs_sh.at[pl.ds(r0, _RPS)])
        plsc.subcore_barrier()

        def group(t, carry):
            j0 = (wid * _CPW + 2 * t) * _CW
            hs = []
            for b in range(2):
                e0 = j0 + b * _CW
                hs.append(pltpu.async_copy(edges_h.at[pl.ds(e0, _CW)],
                                           ebuf.at[b], semi))
                hs.append(pltpu.async_copy(dst_h.at[pl.ds(e0, _CW)],
                                           didx.at[b], semi))
            for h in hs:
                h.wait()
            ss = []
            for b in range(2):
                ss.append(pltpu.async_copy(ebuf.at[b], msgs_sh.at[didx.at[b]],
                                           semsc, add=True))
            for h in ss:
                h.wait()
            return carry

        lax.fori_loop(0, _CPW // 2, group, 0)

        @pl.when(wid < _REM)
        def _():
            e0 = (_NW * _CPW + wid) * _CW
            pltpu.sync_copy(edges_h.at[pl.ds(e0, _CW)], ebuf.at[0])
            pltpu.sync_copy(dst_h.at[pl.ds(e0, _CW)], didx.at[0])
            pltpu.sync_copy(ebuf.at[0], msgs_sh.at[didx.at[0]], add=True)

        plsc.subcore_barrier()
        pltpu.sync_copy(msgs_sh.at[pl.ds(r0, _RPS)],
                        msgs_o.at[cid, pl.ds(r0, _RPS)])

    return k(edges, dst, z128)


def _sc_counts(src, dst, z1, ones1):
    mesh = plsc.VectorSubcoreMesh(core_axis_name="c", subcore_axis_name="s")

    @functools.partial(
        pl.kernel,
        mesh=mesh,
        out_type=[
            jax.ShapeDtypeStruct((_NC * _NP,), jnp.float32),
            jax.ShapeDtypeStruct((_NC * _NP,), jnp.float32),
        ],
        scratch_types=[
            pltpu.VMEM((_CW,), jnp.int32),
            pltpu.VMEM((_CW,), jnp.int32),
            pltpu.VMEM((_CW,), jnp.float32),
            pltpu.VMEM_SHARED((_NP,), jnp.float32),
            pltpu.VMEM_SHARED((_NP,), jnp.float32),
        ],
    )
    def k(src_h, dst_h, z1_h, ones_h, csrc_o, cdst_o,
          sidx, didx, onesv, csrc_sh, cdst_sh):
        cid = lax.axis_index("c")
        sid = lax.axis_index("s")
        wid = sid * _NC + cid
        r0 = sid * _RPS
        pltpu.sync_copy(z1_h.at[pl.ds(r0, _RPS)], csrc_sh.at[pl.ds(r0, _RPS)])
        pltpu.sync_copy(z1_h.at[pl.ds(r0, _RPS)], cdst_sh.at[pl.ds(r0, _RPS)])
        pltpu.sync_copy(ones_h, onesv)
        plsc.subcore_barrier()

        def chunk(k_idx):
            e0 = k_idx * _CW
            pltpu.sync_copy(src_h.at[pl.ds(e0, _CW)], sidx)
            pltpu.sync_copy(dst_h.at[pl.ds(e0, _CW)], didx)
            pltpu.sync_copy(onesv, csrc_sh.at[sidx], add=True)
            pltpu.sync_copy(onesv, cdst_sh.at[didx], add=True)

        def body(j, carry):
            chunk(wid * _CPW + j)
            return carry

        lax.fori_loop(0, _CPW, body, 0)

        @pl.when(wid < _REM)
        def _():
            chunk(_NW * _CPW + wid)

        plsc.subcore_barrier()
        pltpu.sync_copy(csrc_sh.at[pl.ds(r0, _RPS)],
                        csrc_o.at[pl.ds(cid * _NP + r0, _RPS)])
        pltpu.sync_copy(cdst_sh.at[pl.ds(r0, _RPS)],
                        cdst_o.at[pl.ds(cid * _NP + r0, _RPS)])

    return k(src, dst, z1, ones1)


def _sc_gather_add(p, q, src, dst):
    mesh = plsc.VectorSubcoreMesh(core_axis_name="c", subcore_axis_name="s")

    @functools.partial(
        pl.kernel,
        mesh=mesh,
        out_type=jax.ShapeDtypeStruct((_E, _D), jnp.float32),
        scratch_types=[
            pltpu.VMEM((2, _CW), jnp.int32),
            pltpu.VMEM((2, _CW), jnp.int32),
            pltpu.VMEM((2, _CW, _D), jnp.float32),
            pltpu.VMEM((2, _CW, _D), jnp.float32),
            pltpu.SemaphoreType.DMA,
            pltpu.SemaphoreType.DMA,
            pltpu.SemaphoreType.DMA,
        ],
    )
    def k(p_h, q_h, src_h, dst_h, r_o, sidx, didx, pbuf, qbuf,
          semi, semg, sems):
        cid = lax.axis_index("c")
        sid = lax.axis_index("s")
        wid = sid * _NC + cid

        def add_rows(b):
            def row(r, c2):
                for t in range(_D // 16):
                    sl = pl.ds(t * 16, 16)
                    plsc.addupdate(pbuf.at[b, r, sl], qbuf[b, r, sl])
                return c2

            lax.fori_loop(0, _CW, row, 0)

        def group(t, carry):
            j0 = (wid * _CPW + 2 * t) * _CW
            hs = []
            for b in range(2):
                e0 = j0 + b * _CW
                hs.append(pltpu.async_copy(src_h.at[pl.ds(e0, _CW)],
                                           sidx.at[b], semi))
                hs.append(pltpu.async_copy(dst_h.at[pl.ds(e0, _CW)],
                                           didx.at[b], semi))
            for h in hs:
                h.wait()
            gs = []
            for b in range(2):
                gs.append(pltpu.async_copy(p_h.at[sidx.at[b]],
                                           pbuf.at[b], semg))
                gs.append(pltpu.async_copy(q_h.at[didx.at[b]],
                                           qbuf.at[b], semg))
            for h in gs:
                h.wait()
            for b in range(2):
                add_rows(b)
            ss = []
            for b in range(2):
                e0 = j0 + b * _CW
                ss.append(pltpu.async_copy(pbuf.at[b],
                                           r_o.at[pl.ds(e0, _CW)], sems))
            for h in ss:
                h.wait()
            return carry

        lax.fori_loop(0, _CPW // 2, group, 0)

        @pl.when(wid < _REM)
        def _():
            e0 = (_NW * _CPW + wid) * _CW
            pltpu.sync_copy(src_h.at[pl.ds(e0, _CW)], sidx.at[0])
            pltpu.sync_copy(dst_h.at[pl.ds(e0, _CW)], didx.at[0])
            pltpu.async_copy(p_h.at[sidx.at[0]], pbuf.at[0], semg).wait()
            pltpu.async_copy(q_h.at[didx.at[0]], qbuf.at[0], semg).wait()
            add_rows(0)
            pltpu.sync_copy(pbuf.at[0], r_o.at[pl.ds(e0, _CW)])

    return k(p, q, src, dst)


def _esq_body(e_ref, o_ref, acc):
    i = pl.program_id(0)

    @pl.when(i == 0)
    def _():
        acc[...] = jnp.zeros_like(acc)

    x = e_ref[...]
    acc[0:1, :] += jnp.sum(x * x, axis=0, keepdims=True)

    @pl.when(i == pl.num_programs(0) - 1)
    def _():
        o_ref[...] = acc[...]


def _nstats_body(n_ref, mp_ref, msgs_ref, st_ref, acc):
    i = pl.program_id(0)

    @pl.when(i == 0)
    def _():
        acc[...] = jnp.zeros_like(acc)

    x = n_ref[...]
    m = mp_ref[0] + mp_ref[1]
    msgs_ref[...] = m
    acc[0:1, :] += jnp.sum(x, axis=0, keepdims=True)
    acc[1:2, :] += jnp.sum(m, axis=0, keepdims=True)
    acc[2:3, :] += jnp.sum(x * x, axis=0, keepdims=True)
    acc[3:4, :] += jnp.sum(m * m, axis=0, keepdims=True)

    @pl.when(i == pl.num_programs(0) - 1)
    def _():
        st_ref[...] = acc[...]


def _node_body(n_ref, m_ref, w1a_ref, w1b_ref, b1_ref, w2_ref, b2_ref,
               cs_ref, cd_ref, nn_ref, ws_ref, acc):
    i = pl.program_id(0)

    @pl.when(i == 0)
    def _():
        acc[...] = jnp.zeros_like(acc)

    x = n_ref[...]
    m = m_ref[...]
    z = (jnp.dot(x, w1a_ref[...], preferred_element_type=jnp.float32)
         + jnp.dot(m, w1b_ref[...], preferred_element_type=jnp.float32)
         + b1_ref[...])
    h = _gelu(z)
    nn = jnp.dot(h, w2_ref[...], preferred_element_type=jnp.float32) \
        + b2_ref[...] + x
    nn_ref[...] = nn
    nn2 = nn * nn
    cs = cs_ref[...]
    cd = cd_ref[...]
    dn = (((0,), (0,)), ((), ()))
    acc[0:1, :] += lax.dot_general(cs, nn, dn,
                                   preferred_element_type=jnp.float32)
    acc[1:2, :] += lax.dot_general(cs, nn2, dn,
                                   preferred_element_type=jnp.float32)
    acc[2:3, :] += lax.dot_general(cd, nn, dn,
                                   preferred_element_type=jnp.float32)
    acc[3:4, :] += lax.dot_general(cd, nn2, dn,
                                   preferred_element_type=jnp.float32)

    @pl.when(i == pl.num_programs(0) - 1)
    def _():
        ws_ref[...] = acc[...]


def _pq_body(n_ref, a_ref, b_ref, p_ref, q_ref):
    x = n_ref[...]
    p_ref[...] = jnp.dot(x, a_ref[...], preferred_element_type=jnp.float32)
    q_ref[...] = jnp.dot(x, b_ref[...], preferred_element_type=jnp.float32)


def _edge_body(e_ref, r_ref, c_ref, b1_ref, w2_ref, b2_ref, o_ref):
    e = e_ref[...]
    z = jnp.dot(e, c_ref[...], preferred_element_type=jnp.float32) \
        + r_ref[...] + b1_ref[...]
    h = _gelu(z)
    o_ref[...] = jnp.dot(h, w2_ref[...], preferred_element_type=jnp.float32) \
        + b2_ref[...] + e


def _row_spec(shape):
    return pl.BlockSpec(shape, lambda i: (0,) * len(shape))


def kernel(nodes, edges, graph, node_norm_gamma, node_norm_beta,
           edge_norm_gamma, edge_norm_beta,
           nW1, nb1, nW2, nb2, eW1, eb1, eW2, eb2):
    f32 = jnp.float32
    src = graph[0]
    dst = graph[1]
    z128 = jnp.zeros((_NP, _D), f32)
    z1 = jnp.zeros((_NP,), f32)
    ones1 = jnp.ones((_CW,), f32)

    msgs_p = _sc_scatter(edges, dst, z128)
    csrc_1, cdst_1 = _sc_counts(src, dst, z1, ones1)

    sumsq_e = pl.pallas_call(
        _esq_body,
        grid=(_E // _EB,),
        in_specs=[pl.BlockSpec((_EB, _D), lambda i: (i, 0))],
        out_specs=_row_spec((8, _D)),
        out_shape=jax.ShapeDtypeStruct((8, _D), f32),
        scratch_shapes=[pltpu.VMEM((8, _D), f32)],
    )(edges)[0]

    msgs, nst = pl.pallas_call(
        _nstats_body,
        grid=(_N // _NB,),
        in_specs=[
            pl.BlockSpec((_NB, _D), lambda i: (i, 0)),
            pl.BlockSpec((_NC, _NB, _D), lambda i: (0, i, 0)),
        ],
        out_specs=[
            pl.BlockSpec((_NB, _D), lambda i: (i, 0)),
            _row_spec((8, _D)),
        ],
        out_shape=[
            jax.ShapeDtypeStruct((_N, _D), f32),
            jax.ShapeDtypeStruct((8, _D), f32),
        ],
        scratch_shapes=[pltpu.VMEM((8, _D), f32)],
    )(nodes, msgs_p)

    mean_n = jnp.concatenate([nst[0], nst[1]]) / _N
    ex2_n = jnp.concatenate([nst[2], nst[3]]) / _N
    var_n = ex2_n - mean_n * mean_n
    scale_n = node_norm_gamma / jnp.sqrt(var_n + 1e-5)
    shift_n = node_norm_beta - mean_n * scale_n
    w1f = nW1 * scale_n[:, None]
    b1f = (nb1 + shift_n @ nW1).reshape(1, _D)

    csrc = (csrc_1[:_N] + csrc_1[_NP:_NP + _N]).reshape(_N, 1)
    cdst = (cdst_1[:_N] + cdst_1[_NP:_NP + _N]).reshape(_N, 1)

    nodes_new, ws = pl.pallas_call(
        _node_body,
        grid=(_N // _NB,),
        in_specs=[
            pl.BlockSpec((_NB, _D), lambda i: (i, 0)),
            pl.BlockSpec((_NB, _D), lambda i: (i, 0)),
            _row_spec((_D, _D)),
            _row_spec((_D, _D)),
            _row_spec((1, _D)),
            _row_spec((_D, _D)),
            _row_spec((1, _D)),
            pl.BlockSpec((_NB, 1), lambda i: (i, 0)),
            pl.BlockSpec((_NB, 1), lambda i: (i, 0)),
        ],
        out_specs=[
            pl.BlockSpec((_NB, _D), lambda i: (i, 0)),
            _row_spec((8, _D)),
        ],
        out_shape=[
            jax.ShapeDtypeStruct((_N, _D), f32),
            jax.ShapeDtypeStruct((8, _D), f32),
        ],
        scratch_shapes=[pltpu.VMEM((8, _D), f32)],
    )(nodes, msgs, w1f[:_D], w1f[_D:], b1f, nW2, nb2.reshape(1, _D),
      csrc, cdst)

    mean_e = jnp.concatenate([ws[0], ws[2], nst[1]]) / _E
    ex2_e = jnp.concatenate([ws[1], ws[3], sumsq_e]) / _E
    var_e = ex2_e - mean_e * mean_e
    scale_e = edge_norm_gamma / jnp.sqrt(var_e + 1e-5)
    shift_e = edge_norm_beta - mean_e * scale_e
    w1fe = eW1 * scale_e[:, None]
    b1fe = (eb1 + shift_e @ eW1).reshape(1, _D)

    p, q = pl.pallas_call(
        _pq_body,
        grid=(_N // _NB,),
        in_specs=[
            pl.BlockSpec((_NB, _D), lambda i: (i, 0)),
            _row_spec((_D, _D)),
            _row_spec((_D, _D)),
        ],
        out_specs=[
            pl.BlockSpec((_NB, _D), lambda i: (i, 0)),
            pl.BlockSpec((_NB, _D), lambda i: (i, 0)),
        ],
        out_shape=[
            jax.ShapeDtypeStruct((_N, _D), f32),
            jax.ShapeDtypeStruct((_N, _D), f32),
        ],
    )(nodes_new, w1fe[:_D], w1fe[_D:2 * _D])

    r = _sc_gather_add(p, q, src, dst)

    edges_new = pl.pallas_call(
        _edge_body,
        grid=(_E // _EB,),
        in_specs=[
            pl.BlockSpec((_EB, _D), lambda i: (i, 0)),
            pl.BlockSpec((_EB, _D), lambda i: (i, 0)),
            _row_spec((_D, _D)),
            _row_spec((1, _D)),
            _row_spec((_D, _D)),
            _row_spec((1, _D)),
        ],
        out_specs=pl.BlockSpec((_EB, _D), lambda i: (i, 0)),
        out_shape=jax.ShapeDtypeStruct((_E, _D), f32),
    )(edges, r, w1fe[2 * _D:], b1fe, eW2, eb2.reshape(1, _D))

    return (nodes_new, edges_new)


# speedup vs baseline: 4.6141x; 1.1414x over previous
import functools

import jax
import jax.numpy as jnp
from jax import lax
from jax.experimental import pallas as pl
from jax.experimental.pallas import tpu as pltpu
from jax.experimental.pallas import tpu_sc as plsc

_N = 10000
_E = 320000
_D = 128
_NC = 2
_NS = 16
_NW = _NC * _NS
_CW = 128
_NCHUNK = _E // _CW
_CPW = _NCHUNK // _NW
_REM = _NCHUNK - _CPW * _NW
_NP = 10240
_RPS = _NP // _NS
_EB = 2000
_NB = 1000


def _gelu(x):
    return 0.5 * x * (1.0 + lax.erf(x * 0.7071067811865476))


def _sc_scatter(edges, src, dst, z128, z1, ones1):
    mesh = plsc.VectorSubcoreMesh(core_axis_name="c", subcore_axis_name="s")

    @functools.partial(
        pl.kernel,
        mesh=mesh,
        out_type=[
            jax.ShapeDtypeStruct((_NC, _NP, _D), jnp.float32),
            jax.ShapeDtypeStruct((_NC * _NP,), jnp.float32),
            jax.ShapeDtypeStruct((_NC * _NP,), jnp.float32),
        ],
        scratch_types=[
            pltpu.VMEM((2, _CW, _D), jnp.float32),
            pltpu.VMEM((2, _CW), jnp.int32),
            pltpu.VMEM((2, _CW), jnp.int32),
            pltpu.VMEM((_CW,), jnp.float32),
            pltpu.VMEM_SHARED((_NP, _D), jnp.float32),
            pltpu.VMEM_SHARED((_NP,), jnp.float32),
            pltpu.VMEM_SHARED((_NP,), jnp.float32),
            pltpu.SemaphoreType.DMA,
            pltpu.SemaphoreType.DMA,
        ],
    )
    def k(edges_h, src_h, dst_h, z128_h, z1_h, ones_h,
          msgs_o, csrc_o, cdst_o,
          ebuf, sidx, didx, onesv, msgs_sh, csrc_sh, cdst_sh, semi, semsc):
        cid = lax.axis_index("c")
        sid = lax.axis_index("s")
        wid = sid * _NC + cid
        r0 = sid * _RPS
        pltpu.sync_copy(z128_h.at[pl.ds(r0, _RPS)], msgs_sh.at[pl.ds(r0, _RPS)])
        pltpu.sync_copy(z1_h.at[pl.ds(r0, _RPS)], csrc_sh.at[pl.ds(r0, _RPS)])
        pltpu.sync_copy(z1_h.at[pl.ds(r0, _RPS)], cdst_sh.at[pl.ds(r0, _RPS)])
        pltpu.sync_copy(ones_h, onesv)
        plsc.subcore_barrier()

        def group(t, carry):
            j0 = (wid * _CPW + 2 * t) * _CW
            hs = []
            for b in range(2):
                e0 = j0 + b * _CW
                hs.append(pltpu.async_copy(edges_h.at[pl.ds(e0, _CW)],
                                           ebuf.at[b], semi))
                hs.append(pltpu.async_copy(src_h.at[pl.ds(e0, _CW)],
                                           sidx.at[b], semi))
                hs.append(pltpu.async_copy(dst_h.at[pl.ds(e0, _CW)],
                                           didx.at[b], semi))
            for h in hs:
                h.wait()
            ss = []
            for b in range(2):
                ss.append(pltpu.async_copy(ebuf.at[b], msgs_sh.at[didx.at[b]],
                                           semsc, add=True))
                ss.append(pltpu.async_copy(onesv, csrc_sh.at[sidx.at[b]],
                                           semsc, add=True))
                ss.append(pltpu.async_copy(onesv, cdst_sh.at[didx.at[b]],
                                           semsc, add=True))
            for h in ss:
                h.wait()
            return carry

        lax.fori_loop(0, _CPW // 2, group, 0)

        @pl.when(wid < _REM)
        def _():
            e0 = (_NW * _CPW + wid) * _CW
            pltpu.sync_copy(edges_h.at[pl.ds(e0, _CW)], ebuf.at[0])
            pltpu.sync_copy(src_h.at[pl.ds(e0, _CW)], sidx.at[0])
            pltpu.sync_copy(dst_h.at[pl.ds(e0, _CW)], didx.at[0])
            pltpu.sync_copy(ebuf.at[0], msgs_sh.at[didx.at[0]], add=True)
            pltpu.sync_copy(onesv, csrc_sh.at[sidx.at[0]], add=True)
            pltpu.sync_copy(onesv, cdst_sh.at[didx.at[0]], add=True)

        plsc.subcore_barrier()
        pltpu.sync_copy(msgs_sh.at[pl.ds(r0, _RPS)],
                        msgs_o.at[cid, pl.ds(r0, _RPS)])
        pltpu.sync_copy(csrc_sh.at[pl.ds(r0, _RPS)],
                        csrc_o.at[pl.ds(cid * _NP + r0, _RPS)])
        pltpu.sync_copy(cdst_sh.at[pl.ds(r0, _RPS)],
                        cdst_o.at[pl.ds(cid * _NP + r0, _RPS)])

    return k(edges, src, dst, z128, z1, ones1)


def _sc_gather_add(p, q, src, dst):
    mesh = plsc.VectorSubcoreMesh(core_axis_name="c", subcore_axis_name="s")

    @functools.partial(
        pl.kernel,
        mesh=mesh,
        out_type=jax.ShapeDtypeStruct((_E, _D), jnp.float32),
        scratch_types=[
            pltpu.VMEM((2, _CW), jnp.int32),
            pltpu.VMEM((2, _CW), jnp.int32),
            pltpu.VMEM((2, _CW, _D), jnp.float32),
            pltpu.VMEM((2, _CW, _D), jnp.float32),
            pltpu.SemaphoreType.DMA,
            pltpu.SemaphoreType.DMA,
            pltpu.SemaphoreType.DMA,
            pltpu.SemaphoreType.DMA,
        ],
    )
    def k(p_h, q_h, src_h, dst_h, r_o, sidx, didx, pbuf, qbuf,
          semi, semg0, semg1, sems):
        cid = lax.axis_index("c")
        sid = lax.axis_index("s")
        wid = sid * _NC + cid

        def add_rows(b):
            def row(r, c2):
                for t in range(_D // 16):
                    sl = pl.ds(t * 16, 16)
                    plsc.addupdate(pbuf.at[b, r, sl], qbuf[b, r, sl])
                return c2

            lax.fori_loop(0, _CW, row, 0)

        def group(t, carry):
            j0 = (wid * _CPW + 2 * t) * _CW
            hs = []
            for b in range(2):
                e0 = j0 + b * _CW
                hs.append(pltpu.async_copy(src_h.at[pl.ds(e0, _CW)],
                                           sidx.at[b], semi))
                hs.append(pltpu.async_copy(dst_h.at[pl.ds(e0, _CW)],
                                           didx.at[b], semi))
            for h in hs:
                h.wait()
            gsem = (semg0, semg1)
            gs = []
            for b in range(2):
                gs.append(pltpu.async_copy(p_h.at[sidx.at[b]],
                                           pbuf.at[b], gsem[b]))
                gs.append(pltpu.async_copy(q_h.at[didx.at[b]],
                                           qbuf.at[b], gsem[b]))
            ss = []
            for b in range(2):
                gs[2 * b].wait()
                gs[2 * b + 1].wait()
                add_rows(b)
                e0 = j0 + b * _CW
                ss.append(pltpu.async_copy(pbuf.at[b],
                                           r_o.at[pl.ds(e0, _CW)], sems))
            for h in ss:
                h.wait()
            return carry

        lax.fori_loop(0, _CPW // 2, group, 0)

        @pl.when(wid < _REM)
        def _():
            e0 = (_NW * _CPW + wid) * _CW
            pltpu.sync_copy(src_h.at[pl.ds(e0, _CW)], sidx.at[0])
            pltpu.sync_copy(dst_h.at[pl.ds(e0, _CW)], didx.at[0])
            pltpu.async_copy(p_h.at[sidx.at[0]], pbuf.at[0], semg0).wait()
            pltpu.async_copy(q_h.at[didx.at[0]], qbuf.at[0], semg0).wait()
            add_rows(0)
            pltpu.sync_copy(pbuf.at[0], r_o.at[pl.ds(e0, _CW)])

    return k(p, q, src, dst)


def _esq_body(e_ref, o_ref, acc):
    i = pl.program_id(0)

    @pl.when(i == 0)
    def _():
        acc[...] = jnp.zeros_like(acc)

    x = e_ref[...]
    acc[0:1, :] += jnp.sum(x * x, axis=0, keepdims=True)

    @pl.when(i == pl.num_programs(0) - 1)
    def _():
        o_ref[...] = acc[...]


def _nstats_body(n_ref, mp_ref, msgs_ref, st_ref, acc):
    i = pl.program_id(0)

    @pl.when(i == 0)
    def _():
        acc[...] = jnp.zeros_like(acc)

    x = n_ref[...]
    m = mp_ref[0] + mp_ref[1]
    msgs_ref[...] = m
    acc[0:1, :] += jnp.sum(x, axis=0, keepdims=True)
    acc[1:2, :] += jnp.sum(m, axis=0, keepdims=True)
    acc[2:3, :] += jnp.sum(x * x, axis=0, keepdims=True)
    acc[3:4, :] += jnp.sum(m * m, axis=0, keepdims=True)

    @pl.when(i == pl.num_programs(0) - 1)
    def _():
        st_ref[...] = acc[...]


def _node_body(n_ref, m_ref, w1a_ref, w1b_ref, b1_ref, w2_ref, b2_ref,
               cs_ref, cd_ref, nn_ref, ws_ref, acc):
    i = pl.program_id(0)

    @pl.when(i == 0)
    def _():
        acc[...] = jnp.zeros_like(acc)

    x = n_ref[...]
    m = m_ref[...]
    z = (jnp.dot(x, w1a_ref[...], preferred_element_type=jnp.float32)
         + jnp.dot(m, w1b_ref[...], preferred_element_type=jnp.float32)
         + b1_ref[...])
    h = _gelu(z)
    nn = jnp.dot(h, w2_ref[...], preferred_element_type=jnp.float32) \
        + b2_ref[...] + x
    nn_ref[...] = nn
    nn2 = nn * nn
    cs = cs_ref[...]
    cd = cd_ref[...]
    dn = (((0,), (0,)), ((), ()))
    acc[0:1, :] += lax.dot_general(cs, nn, dn,
                                   preferred_element_type=jnp.float32)
    acc[1:2, :] += lax.dot_general(cs, nn2, dn,
                                   preferred_element_type=jnp.float32)
    acc[2:3, :] += lax.dot_general(cd, nn, dn,
                                   preferred_element_type=jnp.float32)
    acc[3:4, :] += lax.dot_general(cd, nn2, dn,
                                   preferred_element_type=jnp.float32)

    @pl.when(i == pl.num_programs(0) - 1)
    def _():
        ws_ref[...] = acc[...]


def _pq_body(n_ref, a_ref, b_ref, p_ref, q_ref):
    x = n_ref[...]
    p_ref[...] = jnp.dot(x, a_ref[...], preferred_element_type=jnp.float32)
    q_ref[...] = jnp.dot(x, b_ref[...], preferred_element_type=jnp.float32)


def _edge_body(e_ref, r_ref, c_ref, b1_ref, w2_ref, b2_ref, o_ref):
    e = e_ref[...]
    z = jnp.dot(e, c_ref[...], preferred_element_type=jnp.float32) \
        + r_ref[...] + b1_ref[...]
    h = _gelu(z)
    o_ref[...] = jnp.dot(h, w2_ref[...], preferred_element_type=jnp.float32) \
        + b2_ref[...] + e


def _row_spec(shape):
    return pl.BlockSpec(shape, lambda i: (0,) * len(shape))


def kernel(nodes, edges, graph, node_norm_gamma, node_norm_beta,
           edge_norm_gamma, edge_norm_beta,
           nW1, nb1, nW2, nb2, eW1, eb1, eW2, eb2):
    f32 = jnp.float32
    src = graph[0]
    dst = graph[1]
    z128 = jnp.zeros((_NP, _D), f32)
    z1 = jnp.zeros((_NP,), f32)
    ones1 = jnp.ones((_CW,), f32)

    msgs_p, csrc_1, cdst_1 = _sc_scatter(edges, src, dst, z128, z1, ones1)

    sumsq_e = pl.pallas_call(
        _esq_body,
        grid=(_E // _EB,),
        in_specs=[pl.BlockSpec((_EB, _D), lambda i: (i, 0))],
        out_specs=_row_spec((8, _D)),
        out_shape=jax.ShapeDtypeStruct((8, _D), f32),
        scratch_shapes=[pltpu.VMEM((8, _D), f32)],
    )(edges)[0]

    msgs, nst = pl.pallas_call(
        _nstats_body,
        grid=(_N // _NB,),
        in_specs=[
            pl.BlockSpec((_NB, _D), lambda i: (i, 0)),
            pl.BlockSpec((_NC, _NB, _D), lambda i: (0, i, 0)),
        ],
        out_specs=[
            pl.BlockSpec((_NB, _D), lambda i: (i, 0)),
            _row_spec((8, _D)),
        ],
        out_shape=[
            jax.ShapeDtypeStruct((_N, _D), f32),
            jax.ShapeDtypeStruct((8, _D), f32),
        ],
        scratch_shapes=[pltpu.VMEM((8, _D), f32)],
    )(nodes, msgs_p)

    mean_n = jnp.concatenate([nst[0], nst[1]]) / _N
    ex2_n = jnp.concatenate([nst[2], nst[3]]) / _N
    var_n = ex2_n - mean_n * mean_n
    scale_n = node_norm_gamma / jnp.sqrt(var_n + 1e-5)
    shift_n = node_norm_beta - mean_n * scale_n
    w1f = nW1 * scale_n[:, None]
    b1f = (nb1 + shift_n @ nW1).reshape(1, _D)

    csrc = (csrc_1[:_N] + csrc_1[_NP:_NP + _N]).reshape(_N, 1)
    cdst = (cdst_1[:_N] + cdst_1[_NP:_NP + _N]).reshape(_N, 1)

    nodes_new, ws = pl.pallas_call(
        _node_body,
        grid=(_N // _NB,),
        in_specs=[
            pl.BlockSpec((_NB, _D), lambda i: (i, 0)),
            pl.BlockSpec((_NB, _D), lambda i: (i, 0)),
            _row_spec((_D, _D)),
            _row_spec((_D, _D)),
            _row_spec((1, _D)),
            _row_spec((_D, _D)),
            _row_spec((1, _D)),
            pl.BlockSpec((_NB, 1), lambda i: (i, 0)),
            pl.BlockSpec((_NB, 1), lambda i: (i, 0)),
        ],
        out_specs=[
            pl.BlockSpec((_NB, _D), lambda i: (i, 0)),
            _row_spec((8, _D)),
        ],
        out_shape=[
            jax.ShapeDtypeStruct((_N, _D), f32),
            jax.ShapeDtypeStruct((8, _D), f32),
        ],
        scratch_shapes=[pltpu.VMEM((8, _D), f32)],
    )(nodes, msgs, w1f[:_D], w1f[_D:], b1f, nW2, nb2.reshape(1, _D),
      csrc, cdst)

    mean_e = jnp.concatenate([ws[0], ws[2], nst[1]]) / _E
    ex2_e = jnp.concatenate([ws[1], ws[3], sumsq_e]) / _E
    var_e = ex2_e - mean_e * mean_e
    scale_e = edge_norm_gamma / jnp.sqrt(var_e + 1e-5)
    shift_e = edge_norm_beta - mean_e * scale_e
    w1fe = eW1 * scale_e[:, None]
    b1fe = (eb1 + shift_e @ eW1).reshape(1, _D)

    p, q = pl.pallas_call(
        _pq_body,
        grid=(_N // _NB,),
        in_specs=[
            pl.BlockSpec((_NB, _D), lambda i: (i, 0)),
            _row_spec((_D, _D)),
            _row_spec((_D, _D)),
        ],
        out_specs=[
            pl.BlockSpec((_NB, _D), lambda i: (i, 0)),
            pl.BlockSpec((_NB, _D), lambda i: (i, 0)),
        ],
        out_shape=[
            jax.ShapeDtypeStruct((_N, _D), f32),
            jax.ShapeDtypeStruct((_N, _D), f32),
        ],
    )(nodes_new, w1fe[:_D], w1fe[_D:2 * _D])

    r = _sc_gather_add(p, q, src, dst)

    edges_new = pl.pallas_call(
        _edge_body,
        grid=(_E // _EB,),
        in_specs=[
            pl.BlockSpec((_EB, _D), lambda i: (i, 0)),
            pl.BlockSpec((_EB, _D), lambda i: (i, 0)),
            _row_spec((_D, _D)),
            _row_spec((1, _D)),
            _row_spec((_D, _D)),
            _row_spec((1, _D)),
        ],
        out_specs=pl.BlockSpec((_EB, _D), lambda i: (i, 0)),
        out_shape=jax.ShapeDtypeStruct((_E, _D), f32),
    )(edges, r, w1fe[2 * _D:], b1fe, eW2, eb2.reshape(1, _D))

    return (nodes_new, edges_new)


# speedup vs baseline: 5.0357x; 1.0914x over previous
import functools

import jax
import jax.numpy as jnp
from jax import lax
from jax.experimental import pallas as pl
from jax.experimental.pallas import tpu as pltpu
from jax.experimental.pallas import tpu_sc as plsc

_N = 10000
_E = 320000
_D = 128
_NC = 2
_NS = 16
_NW = _NC * _NS
_CW = 128
_NCHUNK = _E // _CW
_CPW = _NCHUNK // _NW
_REM = _NCHUNK - _CPW * _NW
_NP = 10240
_RPS = _NP // _NS
_EB = 2000
_NB = 1000


def _gelu(x):
    return 0.5 * x * (1.0 + lax.erf(x * 0.7071067811865476))


def _sc_scatter(edges, src, dst, z128, z1, ones1):
    mesh = plsc.VectorSubcoreMesh(core_axis_name="c", subcore_axis_name="s")

    @functools.partial(
        pl.kernel,
        mesh=mesh,
        out_type=[
            jax.ShapeDtypeStruct((_NC, _NP, _D), jnp.float32),
            jax.ShapeDtypeStruct((_NC * _NP,), jnp.float32),
            jax.ShapeDtypeStruct((_NC * _NP,), jnp.float32),
        ],
        scratch_types=[
            pltpu.VMEM((2, _CW, _D), jnp.float32),
            pltpu.VMEM((2, _CW), jnp.int32),
            pltpu.VMEM((2, _CW), jnp.int32),
            pltpu.VMEM((_CW,), jnp.float32),
            pltpu.VMEM_SHARED((_NP, _D), jnp.float32),
            pltpu.VMEM_SHARED((_NP,), jnp.float32),
            pltpu.VMEM_SHARED((_NP,), jnp.float32),
            pltpu.SemaphoreType.DMA,
            pltpu.SemaphoreType.DMA,
        ],
    )
    def k(edges_h, src_h, dst_h, z128_h, z1_h, ones_h,
          msgs_o, csrc_o, cdst_o,
          ebuf, sidx, didx, onesv, msgs_sh, csrc_sh, cdst_sh, semi, semsc):
        cid = lax.axis_index("c")
        sid = lax.axis_index("s")
        wid = sid * _NC + cid
        r0 = sid * _RPS
        pltpu.sync_copy(z128_h.at[pl.ds(r0, _RPS)], msgs_sh.at[pl.ds(r0, _RPS)])
        pltpu.sync_copy(z1_h.at[pl.ds(r0, _RPS)], csrc_sh.at[pl.ds(r0, _RPS)])
        pltpu.sync_copy(z1_h.at[pl.ds(r0, _RPS)], cdst_sh.at[pl.ds(r0, _RPS)])
        pltpu.sync_copy(ones_h, onesv)
        plsc.subcore_barrier()

        def group(t, carry):
            j0 = (wid * _CPW + 2 * t) * _CW
            hs = []
            for b in range(2):
                e0 = j0 + b * _CW
                hs.append(pltpu.async_copy(edges_h.at[pl.ds(e0, _CW)],
                                           ebuf.at[b], semi))
                hs.append(pltpu.async_copy(src_h.at[pl.ds(e0, _CW)],
                                           sidx.at[b], semi))
                hs.append(pltpu.async_copy(dst_h.at[pl.ds(e0, _CW)],
                                           didx.at[b], semi))
            for h in hs:
                h.wait()
            ss = []
            for b in range(2):
                ss.append(pltpu.async_copy(ebuf.at[b], msgs_sh.at[didx.at[b]],
                                           semsc, add=True))
                ss.append(pltpu.async_copy(onesv, csrc_sh.at[sidx.at[b]],
                                           semsc, add=True))
                ss.append(pltpu.async_copy(onesv, cdst_sh.at[didx.at[b]],
                                           semsc, add=True))
            for h in ss:
                h.wait()
            return carry

        lax.fori_loop(0, _CPW // 2, group, 0)

        @pl.when(wid < _REM)
        def _():
            e0 = (_NW * _CPW + wid) * _CW
            pltpu.sync_copy(edges_h.at[pl.ds(e0, _CW)], ebuf.at[0])
            pltpu.sync_copy(src_h.at[pl.ds(e0, _CW)], sidx.at[0])
            pltpu.sync_copy(dst_h.at[pl.ds(e0, _CW)], didx.at[0])
            pltpu.sync_copy(ebuf.at[0], msgs_sh.at[didx.at[0]], add=True)
            pltpu.sync_copy(onesv, csrc_sh.at[sidx.at[0]], add=True)
            pltpu.sync_copy(onesv, cdst_sh.at[didx.at[0]], add=True)

        plsc.subcore_barrier()
        pltpu.sync_copy(msgs_sh.at[pl.ds(r0, _RPS)],
                        msgs_o.at[cid, pl.ds(r0, _RPS)])
        pltpu.sync_copy(csrc_sh.at[pl.ds(r0, _RPS)],
                        csrc_o.at[pl.ds(cid * _NP + r0, _RPS)])
        pltpu.sync_copy(cdst_sh.at[pl.ds(r0, _RPS)],
                        cdst_o.at[pl.ds(cid * _NP + r0, _RPS)])

    return k(edges, src, dst, z128, z1, ones1)


def _sc_gather_add(p, q, src, dst, c0, nch):
    mesh = plsc.VectorSubcoreMesh(core_axis_name="c", subcore_axis_name="s")
    cpw = nch // _NW
    rem = nch - cpw * _NW

    @functools.partial(
        pl.kernel,
        mesh=mesh,
        out_type=jax.ShapeDtypeStruct((nch * _CW, _D), jnp.float32),
        scratch_types=[
            pltpu.VMEM((2, _CW), jnp.int32),
            pltpu.VMEM((2, _CW), jnp.int32),
            pltpu.VMEM((2, _CW, _D), jnp.float32),
            pltpu.VMEM((2, _CW, _D), jnp.float32),
            pltpu.SemaphoreType.DMA,
            pltpu.SemaphoreType.DMA,
            pltpu.SemaphoreType.DMA,
            pltpu.SemaphoreType.DMA,
        ],
    )
    def k(p_h, q_h, src_h, dst_h, r_o, sidx, didx, pbuf, qbuf,
          semi, semg0, semg1, sems):
        cid = lax.axis_index("c")
        sid = lax.axis_index("s")
        wid = sid * _NC + cid

        def add_rows(b):
            def row(r, c2):
                for t in range(_D // 16):
                    sl = pl.ds(t * 16, 16)
                    plsc.addupdate(pbuf.at[b, r, sl], qbuf[b, r, sl])
                return c2

            lax.fori_loop(0, _CW, row, 0)

        def group(t, carry):
            j0 = (wid * cpw + 2 * t) * _CW
            hs = []
            for b in range(2):
                e0 = j0 + b * _CW
                hs.append(pltpu.async_copy(src_h.at[pl.ds(c0 * _CW + e0, _CW)],
                                           sidx.at[b], semi))
                hs.append(pltpu.async_copy(dst_h.at[pl.ds(c0 * _CW + e0, _CW)],
                                           didx.at[b], semi))
            for h in hs:
                h.wait()
            gsem = (semg0, semg1)
            gs = []
            for b in range(2):
                gs.append(pltpu.async_copy(p_h.at[sidx.at[b]],
                                           pbuf.at[b], gsem[b]))
                gs.append(pltpu.async_copy(q_h.at[didx.at[b]],
                                           qbuf.at[b], gsem[b]))
            ss = []
            for b in range(2):
                gs[2 * b].wait()
                gs[2 * b + 1].wait()
                add_rows(b)
                e0 = j0 + b * _CW
                ss.append(pltpu.async_copy(pbuf.at[b],
                                           r_o.at[pl.ds(e0, _CW)], sems))
            for h in ss:
                h.wait()
            return carry

        lax.fori_loop(0, cpw // 2, group, 0)

        def single(e0):
            pltpu.sync_copy(src_h.at[pl.ds(c0 * _CW + e0, _CW)], sidx.at[0])
            pltpu.sync_copy(dst_h.at[pl.ds(c0 * _CW + e0, _CW)], didx.at[0])
            pltpu.async_copy(p_h.at[sidx.at[0]], pbuf.at[0], semg0).wait()
            pltpu.async_copy(q_h.at[didx.at[0]], qbuf.at[0], semg0).wait()
            add_rows(0)
            pltpu.sync_copy(pbuf.at[0], r_o.at[pl.ds(e0, _CW)])

        if cpw % 2:
            single((wid * cpw + cpw - 1) * _CW)

        @pl.when(wid < rem)
        def _():
            single((_NW * cpw + wid) * _CW)

    return k(p, q, src, dst)


def _esq_body(e_ref, o_ref, acc):
    i = pl.program_id(0)

    @pl.when(i == 0)
    def _():
        acc[...] = jnp.zeros_like(acc)

    x = e_ref[...]
    acc[0:1, :] += jnp.sum(x * x, axis=0, keepdims=True)

    @pl.when(i == pl.num_programs(0) - 1)
    def _():
        o_ref[...] = acc[...]


def _nstats_body(n_ref, mp_ref, msgs_ref, st_ref, acc):
    i = pl.program_id(0)

    @pl.when(i == 0)
    def _():
        acc[...] = jnp.zeros_like(acc)

    x = n_ref[...]
    m = mp_ref[0] + mp_ref[1]
    msgs_ref[...] = m
    acc[0:1, :] += jnp.sum(x, axis=0, keepdims=True)
    acc[1:2, :] += jnp.sum(m, axis=0, keepdims=True)
    acc[2:3, :] += jnp.sum(x * x, axis=0, keepdims=True)
    acc[3:4, :] += jnp.sum(m * m, axis=0, keepdims=True)

    @pl.when(i == pl.num_programs(0) - 1)
    def _():
        st_ref[...] = acc[...]


def _node_body(n_ref, m_ref, w1a_ref, w1b_ref, b1_ref, w2_ref, b2_ref,
               cs_ref, cd_ref, nn_ref, ws_ref, acc):
    i = pl.program_id(0)

    @pl.when(i == 0)
    def _():
        acc[...] = jnp.zeros_like(acc)

    x = n_ref[...]
    m = m_ref[...]
    z = (jnp.dot(x, w1a_ref[...], preferred_element_type=jnp.float32)
         + jnp.dot(m, w1b_ref[...], preferred_element_type=jnp.float32)
         + b1_ref[...])
    h = _gelu(z)
    nn = jnp.dot(h, w2_ref[...], preferred_element_type=jnp.float32) \
        + b2_ref[...] + x
    nn_ref[...] = nn
    nn2 = nn * nn
    cs = cs_ref[...]
    cd = cd_ref[...]
    dn = (((0,), (0,)), ((), ()))
    acc[0:1, :] += lax.dot_general(cs, nn, dn,
                                   preferred_element_type=jnp.float32)
    acc[1:2, :] += lax.dot_general(cs, nn2, dn,
                                   preferred_element_type=jnp.float32)
    acc[2:3, :] += lax.dot_general(cd, nn, dn,
                                   preferred_element_type=jnp.float32)
    acc[3:4, :] += lax.dot_general(cd, nn2, dn,
                                   preferred_element_type=jnp.float32)

    @pl.when(i == pl.num_programs(0) - 1)
    def _():
        ws_ref[...] = acc[...]


def _pq_body(n_ref, a_ref, b_ref, p_ref, q_ref):
    x = n_ref[...]
    p_ref[...] = jnp.dot(x, a_ref[...], preferred_element_type=jnp.float32)
    q_ref[...] = jnp.dot(x, b_ref[...], preferred_element_type=jnp.float32)


def _edge_body(e_ref, r_ref, c_ref, b1_ref, w2_ref, b2_ref, o_ref):
    e = e_ref[...]
    z = jnp.dot(e, c_ref[...], preferred_element_type=jnp.float32) \
        + r_ref[...] + b1_ref[...]
    h = _gelu(z)
    o_ref[...] = jnp.dot(h, w2_ref[...], preferred_element_type=jnp.float32) \
        + b2_ref[...] + e


def _edge_body_alias(e_ref, r_ref, c_ref, b1_ref, w2_ref, b2_ref, a_ref,
                     o_ref):
    del a_ref
    _edge_body(e_ref, r_ref, c_ref, b1_ref, w2_ref, b2_ref, o_ref)


def _row_spec(shape):
    return pl.BlockSpec(shape, lambda i: (0,) * len(shape))


def kernel(nodes, edges, graph, node_norm_gamma, node_norm_beta,
           edge_norm_gamma, edge_norm_beta,
           nW1, nb1, nW2, nb2, eW1, eb1, eW2, eb2):
    f32 = jnp.float32
    src = graph[0]
    dst = graph[1]
    z128 = jnp.zeros((_NP, _D), f32)
    z1 = jnp.zeros((_NP,), f32)
    ones1 = jnp.ones((_CW,), f32)

    msgs_p, csrc_1, cdst_1 = _sc_scatter(edges, src, dst, z128, z1, ones1)

    sumsq_e = pl.pallas_call(
        _esq_body,
        grid=(_E // _EB,),
        in_specs=[pl.BlockSpec((_EB, _D), lambda i: (i, 0))],
        out_specs=_row_spec((8, _D)),
        out_shape=jax.ShapeDtypeStruct((8, _D), f32),
        scratch_shapes=[pltpu.VMEM((8, _D), f32)],
    )(edges)[0]

    msgs, nst = pl.pallas_call(
        _nstats_body,
        grid=(_N // _NB,),
        in_specs=[
            pl.BlockSpec((_NB, _D), lambda i: (i, 0)),
            pl.BlockSpec((_NC, _NB, _D), lambda i: (0, i, 0)),
        ],
        out_specs=[
            pl.BlockSpec((_NB, _D), lambda i: (i, 0)),
            _row_spec((8, _D)),
        ],
        out_shape=[
            jax.ShapeDtypeStruct((_N, _D), f32),
            jax.ShapeDtypeStruct((8, _D), f32),
        ],
        scratch_shapes=[pltpu.VMEM((8, _D), f32)],
    )(nodes, msgs_p)

    mean_n = jnp.concatenate([nst[0], nst[1]]) / _N
    ex2_n = jnp.concatenate([nst[2], nst[3]]) / _N
    var_n = ex2_n - mean_n * mean_n
    scale_n = node_norm_gamma / jnp.sqrt(var_n + 1e-5)
    shift_n = node_norm_beta - mean_n * scale_n
    w1f = nW1 * scale_n[:, None]
    b1f = (nb1 + shift_n @ nW1).reshape(1, _D)

    csrc = (csrc_1[:_N] + csrc_1[_NP:_NP + _N]).reshape(_N, 1)
    cdst = (cdst_1[:_N] + cdst_1[_NP:_NP + _N]).reshape(_N, 1)

    nodes_new, ws = pl.pallas_call(
        _node_body,
        grid=(_N // _NB,),
        in_specs=[
            pl.BlockSpec((_NB, _D), lambda i: (i, 0)),
            pl.BlockSpec((_NB, _D), lambda i: (i, 0)),
            _row_spec((_D, _D)),
            _row_spec((_D, _D)),
            _row_spec((1, _D)),
            _row_spec((_D, _D)),
            _row_spec((1, _D)),
            pl.BlockSpec((_NB, 1), lambda i: (i, 0)),
            pl.BlockSpec((_NB, 1), lambda i: (i, 0)),
        ],
        out_specs=[
            pl.BlockSpec((_NB, _D), lambda i: (i, 0)),
            _row_spec((8, _D)),
        ],
        out_shape=[
            jax.ShapeDtypeStruct((_N, _D), f32),
            jax.ShapeDtypeStruct((8, _D), f32),
        ],
        scratch_shapes=[pltpu.VMEM((8, _D), f32)],
    )(nodes, msgs, w1f[:_D], w1f[_D:], b1f, nW2, nb2.reshape(1, _D),
      csrc, cdst)

    mean_e = jnp.concatenate([ws[0], ws[2], nst[1]]) / _E
    ex2_e = jnp.concatenate([ws[1], ws[3], sumsq_e]) / _E
    var_e = ex2_e - mean_e * mean_e
    scale_e = edge_norm_gamma / jnp.sqrt(var_e + 1e-5)
    shift_e = edge_norm_beta - mean_e * scale_e
    w1fe = eW1 * scale_e[:, None]
    b1fe = (eb1 + shift_e @ eW1).reshape(1, _D)

    p, q = pl.pallas_call(
        _pq_body,
        grid=(_N // _NB,),
        in_specs=[
            pl.BlockSpec((_NB, _D), lambda i: (i, 0)),
            _row_spec((_D, _D)),
            _row_spec((_D, _D)),
        ],
        out_specs=[
            pl.BlockSpec((_NB, _D), lambda i: (i, 0)),
            pl.BlockSpec((_NB, _D), lambda i: (i, 0)),
        ],
        out_shape=[
            jax.ShapeDtypeStruct((_N, _D), f32),
            jax.ShapeDtypeStruct((_N, _D), f32),
        ],
    )(nodes_new, w1fe[:_D], w1fe[_D:2 * _D])

    half = _NCHUNK // 2
    r_a = _sc_gather_add(p, q, src, dst, 0, half)
    r_b = _sc_gather_add(p, q, src, dst, half, _NCHUNK - half)

    c_fold = w1fe[2 * _D:]
    eb2r = eb2.reshape(1, _D)
    hgrid = (_E // 2) // _EB
    hoff = hgrid

    out_a = pl.pallas_call(
        _edge_body,
        grid=(hgrid,),
        in_specs=[
            pl.BlockSpec((_EB, _D), lambda i: (i, 0)),
            pl.BlockSpec((_EB, _D), lambda i: (i, 0)),
            _row_spec((_D, _D)),
            _row_spec((1, _D)),
            _row_spec((_D, _D)),
            _row_spec((1, _D)),
        ],
        out_specs=pl.BlockSpec((_EB, _D), lambda i: (i, 0)),
        out_shape=jax.ShapeDtypeStruct((_E, _D), f32),
    )(edges, r_a, c_fold, b1fe, eW2, eb2r)

    edges_new = pl.pallas_call(
        _edge_body_alias,
        grid=(hgrid,),
        in_specs=[
            pl.BlockSpec((_EB, _D), lambda i: (i + hoff, 0)),
            pl.BlockSpec((_EB, _D), lambda i: (i, 0)),
            _row_spec((_D, _D)),
            _row_spec((1, _D)),
            _row_spec((_D, _D)),
            _row_spec((1, _D)),
            pl.BlockSpec(memory_space=pltpu.MemorySpace.HBM),
        ],
        out_specs=pl.BlockSpec((_EB, _D), lambda i: (i + hoff, 0)),
        out_shape=jax.ShapeDtypeStruct((_E, _D), f32),
        input_output_aliases={6: 0},
    )(edges, r_b, c_fold, b1fe, eW2, eb2r, out_a)

    return (nodes_new, edges_new)


# speedup vs baseline: 5.1403x; 1.0208x over previous
import functools

import jax
import jax.numpy as jnp
from jax import lax
from jax.experimental import pallas as pl
from jax.experimental.pallas import tpu as pltpu
from jax.experimental.pallas import tpu_sc as plsc

_N = 10000
_E = 320000
_D = 128
_NC = 2
_NS = 16
_NW = _NC * _NS
_CW = 128
_NCHUNK = _E // _CW
_CPW = _NCHUNK // _NW
_REM = _NCHUNK - _CPW * _NW
_NP = 10240
_RPS = _NP // _NS
_EB = 2000
_NB = 1000


def _gelu(x):
    return 0.5 * x * (1.0 + lax.erf(x * 0.7071067811865476))


def _sc_scatter(edges, src, dst, z128, z1, ones1):
    mesh = plsc.VectorSubcoreMesh(core_axis_name="c", subcore_axis_name="s")

    @functools.partial(
        pl.kernel,
        mesh=mesh,
        out_type=[
            jax.ShapeDtypeStruct((_NC, _NP, _D), jnp.float32),
            jax.ShapeDtypeStruct((_NC * _NP,), jnp.float32),
            jax.ShapeDtypeStruct((_NC * _NP,), jnp.float32),
        ],
        scratch_types=[
            pltpu.VMEM((2, _CW, _D), jnp.float32),
            pltpu.VMEM((2, _CW), jnp.int32),
            pltpu.VMEM((2, _CW), jnp.int32),
            pltpu.VMEM((_CW,), jnp.float32),
            pltpu.VMEM_SHARED((_NP, _D), jnp.float32),
            pltpu.VMEM_SHARED((_NP,), jnp.float32),
            pltpu.VMEM_SHARED((_NP,), jnp.float32),
            pltpu.SemaphoreType.DMA,
            pltpu.SemaphoreType.DMA,
        ],
    )
    def k(edges_h, src_h, dst_h, z128_h, z1_h, ones_h,
          msgs_o, csrc_o, cdst_o,
          ebuf, sidx, didx, onesv, msgs_sh, csrc_sh, cdst_sh, semi, semsc):
        cid = lax.axis_index("c")
        sid = lax.axis_index("s")
        wid = sid * _NC + cid
        r0 = sid * _RPS
        pltpu.sync_copy(z128_h.at[pl.ds(r0, _RPS)], msgs_sh.at[pl.ds(r0, _RPS)])
        pltpu.sync_copy(z1_h.at[pl.ds(r0, _RPS)], csrc_sh.at[pl.ds(r0, _RPS)])
        pltpu.sync_copy(z1_h.at[pl.ds(r0, _RPS)], cdst_sh.at[pl.ds(r0, _RPS)])
        pltpu.sync_copy(ones_h, onesv)
        plsc.subcore_barrier()

        def group(t, carry):
            j0 = (wid * _CPW + 2 * t) * _CW
            hs = []
            for b in range(2):
                e0 = j0 + b * _CW
                hs.append(pltpu.async_copy(edges_h.at[pl.ds(e0, _CW)],
                                           ebuf.at[b], semi))
                hs.append(pltpu.async_copy(src_h.at[pl.ds(e0, _CW)],
                                           sidx.at[b], semi))
                hs.append(pltpu.async_copy(dst_h.at[pl.ds(e0, _CW)],
                                           didx.at[b], semi))
            for h in hs:
                h.wait()
            ss = []
            for b in range(2):
                ss.append(pltpu.async_copy(ebuf.at[b], msgs_sh.at[didx.at[b]],
                                           semsc, add=True))
                ss.append(pltpu.async_copy(onesv, csrc_sh.at[sidx.at[b]],
                                           semsc, add=True))
                ss.append(pltpu.async_copy(onesv, cdst_sh.at[didx.at[b]],
                                           semsc, add=True))
            for h in ss:
                h.wait()
            return carry

        lax.fori_loop(0, _CPW // 2, group, 0)

        @pl.when(wid < _REM)
        def _():
            e0 = (_NW * _CPW + wid) * _CW
            pltpu.sync_copy(edges_h.at[pl.ds(e0, _CW)], ebuf.at[0])
            pltpu.sync_copy(src_h.at[pl.ds(e0, _CW)], sidx.at[0])
            pltpu.sync_copy(dst_h.at[pl.ds(e0, _CW)], didx.at[0])
            pltpu.sync_copy(ebuf.at[0], msgs_sh.at[didx.at[0]], add=True)
            pltpu.sync_copy(onesv, csrc_sh.at[sidx.at[0]], add=True)
            pltpu.sync_copy(onesv, cdst_sh.at[didx.at[0]], add=True)

        plsc.subcore_barrier()
        pltpu.sync_copy(msgs_sh.at[pl.ds(r0, _RPS)],
                        msgs_o.at[cid, pl.ds(r0, _RPS)])
        pltpu.sync_copy(csrc_sh.at[pl.ds(r0, _RPS)],
                        csrc_o.at[pl.ds(cid * _NP + r0, _RPS)])
        pltpu.sync_copy(cdst_sh.at[pl.ds(r0, _RPS)],
                        cdst_o.at[pl.ds(cid * _NP + r0, _RPS)])

    return k(edges, src, dst, z128, z1, ones1)


def _sc_gather_add(p, q, src, dst, c0, nch):
    mesh = plsc.VectorSubcoreMesh(core_axis_name="c", subcore_axis_name="s")
    cpw = nch // _NW
    rem = nch - cpw * _NW

    @functools.partial(
        pl.kernel,
        mesh=mesh,
        out_type=jax.ShapeDtypeStruct((nch * _CW, _D), jnp.float32),
        scratch_types=[
            pltpu.VMEM((2, _CW), jnp.int32),
            pltpu.VMEM((2, _CW), jnp.int32),
            pltpu.VMEM((2, _CW, _D), jnp.float32),
            pltpu.VMEM((2, _CW, _D), jnp.float32),
            pltpu.SemaphoreType.DMA,
            pltpu.SemaphoreType.DMA,
            pltpu.SemaphoreType.DMA,
            pltpu.SemaphoreType.DMA,
        ],
    )
    def k(p_h, q_h, src_h, dst_h, r_o, sidx, didx, pbuf, qbuf,
          semi, semg0, semg1, sems):
        cid = lax.axis_index("c")
        sid = lax.axis_index("s")
        wid = sid * _NC + cid

        def add_rows(b):
            def row(r, c2):
                for t in range(_D // 16):
                    sl = pl.ds(t * 16, 16)
                    plsc.addupdate(pbuf.at[b, r, sl], qbuf[b, r, sl])
                return c2

            lax.fori_loop(0, _CW, row, 0)

        def group(t, carry):
            j0 = (wid * cpw + 2 * t) * _CW
            hs = []
            for b in range(2):
                e0 = j0 + b * _CW
                hs.append(pltpu.async_copy(src_h.at[pl.ds(c0 * _CW + e0, _CW)],
                                           sidx.at[b], semi))
                hs.append(pltpu.async_copy(dst_h.at[pl.ds(c0 * _CW + e0, _CW)],
                                           didx.at[b], semi))
            for h in hs:
                h.wait()
            gsem = (semg0, semg1)
            gs = []
            for b in range(2):
                gs.append(pltpu.async_copy(p_h.at[sidx.at[b]],
                                           pbuf.at[b], gsem[b]))
                gs.append(pltpu.async_copy(q_h.at[didx.at[b]],
                                           qbuf.at[b], gsem[b]))
            ss = []
            for b in range(2):
                gs[2 * b].wait()
                gs[2 * b + 1].wait()
                add_rows(b)
                e0 = j0 + b * _CW
                ss.append(pltpu.async_copy(pbuf.at[b],
                                           r_o.at[pl.ds(e0, _CW)], sems))
            for h in ss:
                h.wait()
            return carry

        lax.fori_loop(0, cpw // 2, group, 0)

        def single(e0):
            pltpu.sync_copy(src_h.at[pl.ds(c0 * _CW + e0, _CW)], sidx.at[0])
            pltpu.sync_copy(dst_h.at[pl.ds(c0 * _CW + e0, _CW)], didx.at[0])
            pltpu.async_copy(p_h.at[sidx.at[0]], pbuf.at[0], semg0).wait()
            pltpu.async_copy(q_h.at[didx.at[0]], qbuf.at[0], semg0).wait()
            add_rows(0)
            pltpu.sync_copy(pbuf.at[0], r_o.at[pl.ds(e0, _CW)])

        if cpw % 2:
            single((wid * cpw + cpw - 1) * _CW)

        @pl.when(wid < rem)
        def _():
            single((_NW * cpw + wid) * _CW)

    return k(p, q, src, dst)


def _esq_body(e_ref, o_ref, acc):
    i = pl.program_id(0)

    @pl.when(i == 0)
    def _():
        acc[...] = jnp.zeros_like(acc)

    x = e_ref[...]
    acc[0:1, :] += jnp.sum(x * x, axis=0, keepdims=True)

    @pl.when(i == pl.num_programs(0) - 1)
    def _():
        o_ref[...] = acc[...]


def _nstats_body(n_ref, mp_ref, msgs_ref, st_ref, acc):
    i = pl.program_id(0)

    @pl.when(i == 0)
    def _():
        acc[...] = jnp.zeros_like(acc)

    x = n_ref[...]
    m = mp_ref[0] + mp_ref[1]
    msgs_ref[...] = m
    acc[0:1, :] += jnp.sum(x, axis=0, keepdims=True)
    acc[1:2, :] += jnp.sum(m, axis=0, keepdims=True)
    acc[2:3, :] += jnp.sum(x * x, axis=0, keepdims=True)
    acc[3:4, :] += jnp.sum(m * m, axis=0, keepdims=True)

    @pl.when(i == pl.num_programs(0) - 1)
    def _():
        st_ref[...] = acc[...]


def _node_body(n_ref, m_ref, w1a_ref, w1b_ref, b1_ref, w2_ref, b2_ref,
               cs_ref, cd_ref, nn_ref, ws_ref, acc):
    i = pl.program_id(0)

    @pl.when(i == 0)
    def _():
        acc[...] = jnp.zeros_like(acc)

    x = n_ref[...]
    m = m_ref[...]
    z = (jnp.dot(x, w1a_ref[...], preferred_element_type=jnp.float32)
         + jnp.dot(m, w1b_ref[...], preferred_element_type=jnp.float32)
         + b1_ref[...])
    h = _gelu(z)
    nn = jnp.dot(h, w2_ref[...], preferred_element_type=jnp.float32) \
        + b2_ref[...] + x
    nn_ref[...] = nn
    nn2 = nn * nn
    cs = cs_ref[...]
    cd = cd_ref[...]
    dn = (((0,), (0,)), ((), ()))
    acc[0:1, :] += lax.dot_general(cs, nn, dn,
                                   preferred_element_type=jnp.float32)
    acc[1:2, :] += lax.dot_general(cs, nn2, dn,
                                   preferred_element_type=jnp.float32)
    acc[2:3, :] += lax.dot_general(cd, nn, dn,
                                   preferred_element_type=jnp.float32)
    acc[3:4, :] += lax.dot_general(cd, nn2, dn,
                                   preferred_element_type=jnp.float32)

    @pl.when(i == pl.num_programs(0) - 1)
    def _():
        ws_ref[...] = acc[...]


def _pq_body(n_ref, a_ref, b_ref, p_ref, q_ref):
    x = n_ref[...]
    p_ref[...] = jnp.dot(x, a_ref[...], preferred_element_type=jnp.float32)
    q_ref[...] = jnp.dot(x, b_ref[...], preferred_element_type=jnp.float32)


def _edge_body(e_ref, r_ref, c_ref, b1_ref, w2_ref, b2_ref, o_ref):
    e = e_ref[...]
    z = jnp.dot(e, c_ref[...], preferred_element_type=jnp.float32) \
        + r_ref[...] + b1_ref[...]
    h = _gelu(z)
    o_ref[...] = jnp.dot(h, w2_ref[...], preferred_element_type=jnp.float32) \
        + b2_ref[...] + e


def _edge_body_alias(e_ref, r_ref, c_ref, b1_ref, w2_ref, b2_ref, a_ref,
                     o_ref):
    del a_ref
    _edge_body(e_ref, r_ref, c_ref, b1_ref, w2_ref, b2_ref, o_ref)


def _row_spec(shape):
    return pl.BlockSpec(shape, lambda i: (0,) * len(shape))


def kernel(nodes, edges, graph, node_norm_gamma, node_norm_beta,
           edge_norm_gamma, edge_norm_beta,
           nW1, nb1, nW2, nb2, eW1, eb1, eW2, eb2):
    f32 = jnp.float32
    src = graph[0]
    dst = graph[1]
    z128 = jnp.zeros((_NP, _D), f32)
    z1 = jnp.zeros((_NP,), f32)
    ones1 = jnp.ones((_CW,), f32)

    msgs_p, csrc_1, cdst_1 = _sc_scatter(edges, src, dst, z128, z1, ones1)

    sumsq_e = pl.pallas_call(
        _esq_body,
        grid=(_E // _EB,),
        in_specs=[pl.BlockSpec((_EB, _D), lambda i: (i, 0))],
        out_specs=_row_spec((8, _D)),
        out_shape=jax.ShapeDtypeStruct((8, _D), f32),
        scratch_shapes=[pltpu.VMEM((8, _D), f32)],
    )(edges)[0]

    msgs, nst = pl.pallas_call(
        _nstats_body,
        grid=(_N // _NB,),
        in_specs=[
            pl.BlockSpec((_NB, _D), lambda i: (i, 0)),
            pl.BlockSpec((_NC, _NB, _D), lambda i: (0, i, 0)),
        ],
        out_specs=[
            pl.BlockSpec((_NB, _D), lambda i: (i, 0)),
            _row_spec((8, _D)),
        ],
        out_shape=[
            jax.ShapeDtypeStruct((_N, _D), f32),
            jax.ShapeDtypeStruct((8, _D), f32),
        ],
        scratch_shapes=[pltpu.VMEM((8, _D), f32)],
    )(nodes, msgs_p)

    mean_n = jnp.concatenate([nst[0], nst[1]]) / _N
    ex2_n = jnp.concatenate([nst[2], nst[3]]) / _N
    var_n = ex2_n - mean_n * mean_n
    scale_n = node_norm_gamma / jnp.sqrt(var_n + 1e-5)
    shift_n = node_norm_beta - mean_n * scale_n
    w1f = nW1 * scale_n[:, None]
    b1f = (nb1 + shift_n @ nW1).reshape(1, _D)

    csrc = (csrc_1[:_N] + csrc_1[_NP:_NP + _N]).reshape(_N, 1)
    cdst = (cdst_1[:_N] + cdst_1[_NP:_NP + _N]).reshape(_N, 1)

    nodes_new, ws = pl.pallas_call(
        _node_body,
        grid=(_N // _NB,),
        in_specs=[
            pl.BlockSpec((_NB, _D), lambda i: (i, 0)),
            pl.BlockSpec((_NB, _D), lambda i: (i, 0)),
            _row_spec((_D, _D)),
            _row_spec((_D, _D)),
            _row_spec((1, _D)),
            _row_spec((_D, _D)),
            _row_spec((1, _D)),
            pl.BlockSpec((_NB, 1), lambda i: (i, 0)),
            pl.BlockSpec((_NB, 1), lambda i: (i, 0)),
        ],
        out_specs=[
            pl.BlockSpec((_NB, _D), lambda i: (i, 0)),
            _row_spec((8, _D)),
        ],
        out_shape=[
            jax.ShapeDtypeStruct((_N, _D), f32),
            jax.ShapeDtypeStruct((8, _D), f32),
        ],
        scratch_shapes=[pltpu.VMEM((8, _D), f32)],
    )(nodes, msgs, w1f[:_D], w1f[_D:], b1f, nW2, nb2.reshape(1, _D),
      csrc, cdst)

    mean_e = jnp.concatenate([ws[0], ws[2], nst[1]]) / _E
    ex2_e = jnp.concatenate([ws[1], ws[3], sumsq_e]) / _E
    var_e = ex2_e - mean_e * mean_e
    scale_e = edge_norm_gamma / jnp.sqrt(var_e + 1e-5)
    shift_e = edge_norm_beta - mean_e * scale_e
    w1fe = eW1 * scale_e[:, None]
    b1fe = (eb1 + shift_e @ eW1).reshape(1, _D)

    p, q = pl.pallas_call(
        _pq_body,
        grid=(_N // _NB,),
        in_specs=[
            pl.BlockSpec((_NB, _D), lambda i: (i, 0)),
            _row_spec((_D, _D)),
            _row_spec((_D, _D)),
        ],
        out_specs=[
            pl.BlockSpec((_NB, _D), lambda i: (i, 0)),
            pl.BlockSpec((_NB, _D), lambda i: (i, 0)),
        ],
        out_shape=[
            jax.ShapeDtypeStruct((_N, _D), f32),
            jax.ShapeDtypeStruct((_N, _D), f32),
        ],
    )(nodes_new, w1fe[:_D], w1fe[_D:2 * _D])

    nseg = 4
    seg = _NCHUNK // nseg
    rs = [_sc_gather_add(p, q, src, dst, s * seg, seg) for s in range(nseg)]

    c_fold = w1fe[2 * _D:]
    eb2r = eb2.reshape(1, _D)
    hgrid = seg * _CW // _EB

    out = None
    for s in range(nseg):
        off = s * hgrid
        common = [
            pl.BlockSpec((_EB, _D), lambda i, off=off: (i + off, 0)),
            pl.BlockSpec((_EB, _D), lambda i: (i, 0)),
            _row_spec((_D, _D)),
            _row_spec((1, _D)),
            _row_spec((_D, _D)),
            _row_spec((1, _D)),
        ]
        if s == 0:
            out = pl.pallas_call(
                _edge_body,
                grid=(hgrid,),
                in_specs=common,
                out_specs=pl.BlockSpec((_EB, _D), lambda i: (i, 0)),
                out_shape=jax.ShapeDtypeStruct((_E, _D), f32),
            )(edges, rs[0], c_fold, b1fe, eW2, eb2r)
        else:
            out = pl.pallas_call(
                _edge_body_alias,
                grid=(hgrid,),
                in_specs=common + [
                    pl.BlockSpec(memory_space=pltpu.MemorySpace.HBM)],
                out_specs=pl.BlockSpec((_EB, _D),
                                       lambda i, off=off: (i + off, 0)),
                out_shape=jax.ShapeDtypeStruct((_E, _D), f32),
                input_output_aliases={6: 0},
            )(edges, rs[s], c_fold, b1fe, eW2, eb2r, out)

    return (nodes_new, out)
